# Initial kernel scaffold; baseline (speedup 1.0000x reference)
import functools

import jax
import jax.numpy as jnp
from jax import lax
from jax.experimental import pallas as pl
from jax.experimental.pallas import tpu as pltpu
from jax.experimental.pallas import tpu_sc as plsc

N_NODES = 16384
N_EDGES = 65536
D_IN = 128
D = 32
EL = 128
MIX_N = 10
MAX_N = 64
BN_EPS = 1e-5
VAR_EPS = 1e-5

NW = 32
EPW = N_EDGES // NW
CHUNK = 128
NCH = EPW // CHUNK


def _embed_body(x_ref, wt_ref, b_ref, g_ref, bt_ref, y_ref):
    y = jnp.dot(x_ref[...], wt_ref[...], preferred_element_type=jnp.float32)
    y = y + b_ref[...]
    mean = jnp.mean(y, axis=0, keepdims=True)
    var = jnp.mean((y - mean) * (y - mean), axis=0, keepdims=True)
    y_ref[...] = (y - mean) * lax.rsqrt(var + BN_EPS) * g_ref[...] + bt_ref[...]


def _embed(x, wt, b, g, bt):
    return pl.pallas_call(
        _embed_body,
        out_shape=jax.ShapeDtypeStruct((N_NODES, D), jnp.float32),
    )(x, wt, b, g, bt)


def _edges_body(ea_ref, w1t_ref, b1_ref, eh_ref):
    t = jnp.dot(ea_ref[...], w1t_ref[...], preferred_element_type=jnp.float32)
    eh_ref[...] = jax.nn.sigmoid(t + b1_ref[...]).astype(jnp.bfloat16)


def _edges(ea, w1t, b1):
    eb = 8192
    return pl.pallas_call(
        _edges_body,
        grid=(N_EDGES // eb,),
        in_specs=[
            pl.BlockSpec((eb, 4), lambda i: (i, 0)),
            pl.BlockSpec((4, EL), lambda i: (0, 0)),
            pl.BlockSpec((1, EL), lambda i: (0, 0)),
        ],
        out_specs=pl.BlockSpec((eb, EL), lambda i: (i, 0)),
        out_shape=jax.ShapeDtypeStruct((N_EDGES, EL), jnp.bfloat16),
    )(ea, w1t, b1)


def _msg_body(eh_ref, cs_ref, w2_ref, b2_ref, t_ref, g_ref, msg_ref):
    wg = jnp.dot(eh_ref[...], w2_ref[...], preferred_element_type=jnp.float32)
    wg = wg + b2_ref[...]
    ct = jnp.dot(cs_ref[...], t_ref[...], preferred_element_type=jnp.float32)
    msg_ref[...] = jnp.dot(wg * ct, g_ref[...], preferred_element_type=jnp.float32)


def _msg(eh, cs, w2t, b2p, tmat, gmat):
    eb = 1024
    return pl.pallas_call(
        _msg_body,
        grid=(N_EDGES // eb,),
        in_specs=[
            pl.BlockSpec((eb, EL), lambda i: (i, 0)),
            pl.BlockSpec((eb, D), lambda i: (i, 0)),
            pl.BlockSpec((EL, D * D), lambda i: (0, 0)),
            pl.BlockSpec((1, D * D), lambda i: (0, 0)),
            pl.BlockSpec((D, D * D), lambda i: (0, 0)),
            pl.BlockSpec((D * D, D), lambda i: (0, 0)),
        ],
        out_specs=pl.BlockSpec((eb, D), lambda i: (i, 0)),
        out_shape=jax.ShapeDtypeStruct((N_EDGES, D), jnp.float32),
    )(eh, cs, w2t, b2p, tmat, gmat)


def _upd_body(a0_ref, a1_ref, c0_ref, c1_ref, s_ref, root_ref, cb_ref,
              wri_ref, wzi_ref, wni_ref, wrh_ref, wzh_ref, wnh_ref,
              bi_ref, bh_ref, out_ref):
    cnt = jnp.maximum(c0_ref[...][:, :1] + c1_ref[...][:, :1], 1.0)
    agg = (a0_ref[...] + a1_ref[...]) / cnt
    s = s_ref[...]
    m = agg + jnp.dot(s, root_ref[...], preferred_element_type=jnp.float32)
    m = jnp.maximum(m + cb_ref[...], 0.0)
    bi = bi_ref[...]
    bh = bh_ref[...]
    gir = jnp.dot(m, wri_ref[...], preferred_element_type=jnp.float32) + bi[:, :D]
    giz = jnp.dot(m, wzi_ref[...], preferred_element_type=jnp.float32) + bi[:, D:2 * D]
    gin = jnp.dot(m, wni_ref[...], preferred_element_type=jnp.float32) + bi[:, 2 * D:]
    ghr = jnp.dot(s, wrh_ref[...], preferred_element_type=jnp.float32) + bh[:, :D]
    ghz = jnp.dot(s, wzh_ref[...], preferred_element_type=jnp.float32) + bh[:, D:2 * D]
    ghn = jnp.dot(s, wnh_ref[...], preferred_element_type=jnp.float32) + bh[:, 2 * D:]
    r = jax.nn.sigmoid(gir + ghr)
    z = jax.nn.sigmoid(giz + ghz)
    n = jnp.tanh(gin + r * ghn)
    out_ref[...] = (1.0 - z) * n + z * s


def _update(aggp, cntp, s, root, cb, wsplits, bi, bh):
    nb = 4096
    specs = [pl.BlockSpec((nb, D), lambda i: (i, 0)) for _ in range(5)]
    wspec = [pl.BlockSpec((D, D), lambda i: (0, 0)) for _ in range(7)]
    bspec = [pl.BlockSpec((1, 3 * D), lambda i: (0, 0)) for _ in range(2)]
    cbspec = [pl.BlockSpec((1, D), lambda i: (0, 0))]
    return pl.pallas_call(
        _upd_body,
        grid=(N_NODES // nb,),
        in_specs=specs + wspec[:1] + cbspec + wspec[1:] + bspec,
        out_specs=pl.BlockSpec((nb, D), lambda i: (i, 0)),
        out_shape=jax.ShapeDtypeStruct((N_NODES, D), jnp.float32),
    )(aggp[0], aggp[1], cntp[0], cntp[1], s, root, cb, *wsplits, bi, bh)


def _head_body(s_ref, w1_ref, b1_ref, w2_ref, b2_ref, w3_ref, b3_ref,
               sel_ref, mu_ref, std_ref):
    t1 = jnp.dot(s_ref[...], w1_ref[...], preferred_element_type=jnp.float32)
    t1 = jnp.maximum(t1 + b1_ref[...], 0.0)
    t2 = jnp.dot(t1, w2_ref[...], preferred_element_type=jnp.float32)
    t2 = jnp.maximum(t2 + b2_ref[...], 0.0)
    mix = jnp.dot(t2, w3_ref[...], preferred_element_type=jnp.float32) + b3_ref[...]
    sel = sel_ref[...]
    lane = lax.broadcasted_iota(jnp.int32, mix.shape, 1)
    oh = (lane == sel).astype(jnp.float32)
    mu_ref[...] = jnp.sum(mix * oh, axis=1, keepdims=True)
    mm = jnp.mean(mix, axis=1, keepdims=True)
    var = jnp.sum((mix - mm) * (mix - mm), axis=1, keepdims=True) / (MIX_N - 1.0)
    std_ref[...] = jnp.sqrt(var + VAR_EPS)


def _heads(s, w1, b1, w2, b2, w3, b3, sel):
    nb = 4096
    kd = MIX_N * D
    return pl.pallas_call(
        _head_body,
        grid=(N_NODES // nb,),
        in_specs=[
            pl.BlockSpec((nb, D), lambda i: (i, 0)),
            pl.BlockSpec((D, kd), lambda i: (0, 0)),
            pl.BlockSpec((1, kd), lambda i: (0, 0)),
            pl.BlockSpec((kd, kd), lambda i: (0, 0)),
            pl.BlockSpec((1, kd), lambda i: (0, 0)),
            pl.BlockSpec((kd, MIX_N), lambda i: (0, 0)),
            pl.BlockSpec((1, MIX_N), lambda i: (0, 0)),
            pl.BlockSpec((nb, 1), lambda i: (i, 0)),
        ],
        out_specs=[
            pl.BlockSpec((nb, 1), lambda i: (i, 0)),
            pl.BlockSpec((nb, 1), lambda i: (i, 0)),
        ],
        out_shape=[
            jax.ShapeDtypeStruct((N_NODES, 1), jnp.float32),
            jax.ShapeDtypeStruct((N_NODES, 1), jnp.float32),
        ],
    )(s, w1, b1, w2, b2, w3, b3, sel)


def _gather_body(cur_hbm, src_hbm, out_hbm, idx_v, rows_v, sem):
    c = lax.axis_index("c")
    s = lax.axis_index("s")
    wid = s * 2 + c
    pltpu.sync_copy(src_hbm.at[pl.ds(wid * NCH, NCH)], idx_v)
    copies = []
    for j in range(NCH):
        copies.append(pltpu.async_copy(
            cur_hbm.at[idx_v.at[j]], rows_v.at[pl.ds(j * CHUNK, CHUNK)], sem))
    for cp in copies:
        cp.wait()
    pltpu.sync_copy(rows_v, out_hbm.at[pl.ds(wid * EPW, EPW)])


def _sc_gather(cur, src2d):
    mesh = plsc.VectorSubcoreMesh(core_axis_name="c", subcore_axis_name="s")
    return pl.kernel(
        _gather_body,
        out_type=jax.ShapeDtypeStruct((N_EDGES, D), jnp.float32),
        mesh=mesh,
        scratch_types=[
            pltpu.VMEM((NCH, CHUNK), jnp.int32),
            pltpu.VMEM((EPW, D), jnp.float32),
            pltpu.SemaphoreType.DMA,
        ],
    )(cur, src2d)


def _scatter_body(msg_hbm, dst_hbm, zero_hbm, out_hbm, idx_v, rows_v, acc_sh, sem):
    c = lax.axis_index("c")
    s = lax.axis_index("s")
    wid = s * 2 + c
    rows_per_sub = N_NODES // 16
    pltpu.sync_copy(zero_hbm, acc_sh.at[pl.ds(s * rows_per_sub, rows_per_sub)])
    pltpu.sync_copy(dst_hbm.at[pl.ds(wid * NCH, NCH)], idx_v)
    pltpu.sync_copy(msg_hbm.at[pl.ds(wid * EPW, EPW)], rows_v)
    plsc.subcore_barrier()
    for j in range(NCH):
        pltpu.sync_copy(rows_v.at[pl.ds(j * CHUNK, CHUNK)],
                        acc_sh.at[idx_v.at[j]], add=True)
    plsc.subcore_barrier()
    pltpu.sync_copy(acc_sh.at[pl.ds(s * rows_per_sub, rows_per_sub)],
                    out_hbm.at[c, pl.ds(s * rows_per_sub, rows_per_sub)])


def _sc_scatter(msg, dst2d, zero_rows):
    mesh = plsc.VectorSubcoreMesh(core_axis_name="c", subcore_axis_name="s")
    return pl.kernel(
        _scatter_body,
        out_type=jax.ShapeDtypeStruct((2, N_NODES, D), jnp.float32),
        mesh=mesh,
        scratch_types=[
            pltpu.VMEM((NCH, CHUNK), jnp.int32),
            pltpu.VMEM((EPW, D), jnp.float32),
            pltpu.VMEM_SHARED((N_NODES, D), jnp.float32),
            pltpu.SemaphoreType.DMA,
        ],
    )(msg, dst2d, zero_rows)


def kernel(x, edge_index, edge_attr, input_idx, W_embed, b_embed, bn_gamma,
           bn_beta, We1, be1, We2, be2, root, conv_bias, Wih, Whh, bih, bhh,
           mW1, mb1, mW2, mb2, mW3, mb3):
    f32 = jnp.float32
    src2d = edge_index[0].reshape(N_EDGES // CHUNK, CHUNK)
    dst2d = edge_index[1].reshape(N_EDGES // CHUNK, CHUNK)

    wt = W_embed.T
    b2 = b_embed.reshape(1, D)
    g2 = bn_gamma.reshape(1, D)
    bt2 = bn_beta.reshape(1, D)
    w1t = We1.T
    be1r = be1.reshape(1, EL)
    we2p = We2.reshape(D, D, EL).transpose(1, 0, 2).reshape(D * D, EL)
    w2t = we2p.T.astype(jnp.bfloat16)
    be2p = be2.reshape(D, D).T.reshape(1, D * D)
    eye = jnp.eye(D, dtype=f32)
    tmat = jnp.tile(eye, (1, D))
    gmat = jnp.repeat(eye, D, axis=0)
    cb = conv_bias.reshape(1, D)
    wsplits = (Wih[:D].T, Wih[D:2 * D].T, Wih[2 * D:].T,
               Whh[:D].T, Whh[D:2 * D].T, Whh[2 * D:].T)
    bi = bih.reshape(1, 3 * D)
    bh = bhh.reshape(1, 3 * D)
    kd = MIX_N * D
    w1 = jnp.transpose(mW1, (2, 0, 1)).reshape(D, kd)
    b1 = mb1.reshape(1, kd)
    w2bd = jax.scipy.linalg.block_diag(*[mW2[k].T for k in range(MIX_N)])
    b2h = mb2.reshape(1, kd)
    w3bd = jax.scipy.linalg.block_diag(*[mW3[k].T for k in range(MIX_N)])
    b3h = mb3.reshape(1, MIX_N)
    sel = jnp.repeat(jnp.mod(input_idx, MIX_N), MAX_N).reshape(N_NODES, 1)
    sel = sel.astype(jnp.int32)
    zero_rows = jnp.zeros((N_NODES // 16, D), f32)
    ones_rows = jnp.ones((N_EDGES, D), f32)

    y = _embed(x, wt, b2, g2, bt2)
    eh = _edges(edge_attr, w1t, be1r)
    cntp = _sc_scatter(ones_rows, dst2d, zero_rows)

    s = y
    for _ in range(3):
        cs = _sc_gather(s, src2d)
        msg = _msg(eh, cs, w2t, be2p, tmat, gmat)
        aggp = _sc_scatter(msg, dst2d, zero_rows)
        s = _update(aggp, cntp, s, root, cb, wsplits, bi, bh)

    mu, std = _heads(s, w1, b1, w2bd, b2h, w3bd, b3h, sel)
    return (mu.reshape(N_NODES // MAX_N, MAX_N, 1),
            std.reshape(N_NODES // MAX_N, MAX_N, 1))


# speedup vs baseline: 1.9197x; 1.9197x over previous
import functools

import jax
import jax.numpy as jnp
from jax import lax
from jax.experimental import pallas as pl
from jax.experimental.pallas import tpu as pltpu
from jax.experimental.pallas import tpu_sc as plsc

N_NODES = 16384
N_EDGES = 65536
D_IN = 128
D = 32
EL = 128
MIX_N = 10
MAX_N = 64
BN_EPS = 1e-5
VAR_EPS = 1e-5

NW = 32
EPW = N_EDGES // NW
CHUNK = 128
NCH = EPW // CHUNK


def _embed_body(x_ref, wt_ref, b_ref, g_ref, bt_ref, y_ref):
    y = jnp.dot(x_ref[...], wt_ref[...], preferred_element_type=jnp.float32)
    y = y + b_ref[...]
    mean = jnp.mean(y, axis=0, keepdims=True)
    var = jnp.mean((y - mean) * (y - mean), axis=0, keepdims=True)
    y_ref[...] = (y - mean) * lax.rsqrt(var + BN_EPS) * g_ref[...] + bt_ref[...]


def _embed(x, wt, b, g, bt):
    return pl.pallas_call(
        _embed_body,
        out_shape=jax.ShapeDtypeStruct((N_NODES, D), jnp.float32),
    )(x, wt, b, g, bt)


def _edges_body(ea_ref, w1t_ref, b1_ref, eh_ref):
    t = jnp.dot(ea_ref[...], w1t_ref[...], preferred_element_type=jnp.float32)
    eh_ref[...] = jax.nn.sigmoid(t + b1_ref[...]).astype(jnp.bfloat16)


def _edges(ea, w1t, b1):
    eb = 8192
    return pl.pallas_call(
        _edges_body,
        grid=(N_EDGES // eb,),
        in_specs=[
            pl.BlockSpec((eb, 4), lambda i: (i, 0)),
            pl.BlockSpec((4, EL), lambda i: (0, 0)),
            pl.BlockSpec((1, EL), lambda i: (0, 0)),
        ],
        out_specs=pl.BlockSpec((eb, EL), lambda i: (i, 0)),
        out_shape=jax.ShapeDtypeStruct((N_EDGES, EL), jnp.bfloat16),
    )(ea, w1t, b1)


def _msg_body(eh_ref, csw_ref, sq_ref, dq_ref, w2_ref, b2_ref, t_ref, g_ref,
              msg_ref):
    f32 = jnp.float32
    sq = sq_ref[...]
    csw = csw_ref[...]
    cs = ((sq == 0).astype(f32) * csw[:, 0 * D:1 * D]
          + (sq == 1).astype(f32) * csw[:, 1 * D:2 * D]
          + (sq == 2).astype(f32) * csw[:, 2 * D:3 * D]
          + (sq == 3).astype(f32) * csw[:, 3 * D:4 * D])
    wg = jnp.dot(eh_ref[...], w2_ref[...], preferred_element_type=f32)
    wg = wg + b2_ref[...]
    ct = jnp.dot(cs, t_ref[...], preferred_element_type=f32)
    msg = jnp.dot(wg * ct, g_ref[...], preferred_element_type=f32)
    dq = dq_ref[...]
    msg_ref[...] = jnp.concatenate(
        [msg * (dq == 0).astype(f32), msg * (dq == 1).astype(f32),
         msg * (dq == 2).astype(f32), msg * (dq == 3).astype(f32)], axis=1)


def _msg(eh, csw, sq, dq, w2t, b2p, tmat, gmat):
    eb = 1024
    return pl.pallas_call(
        _msg_body,
        grid=(N_EDGES // eb,),
        in_specs=[
            pl.BlockSpec((eb, EL), lambda i: (i, 0)),
            pl.BlockSpec((eb, 4 * D), lambda i: (i, 0)),
            pl.BlockSpec((eb, 1), lambda i: (i, 0)),
            pl.BlockSpec((eb, 1), lambda i: (i, 0)),
            pl.BlockSpec((EL, D * D), lambda i: (0, 0)),
            pl.BlockSpec((1, D * D), lambda i: (0, 0)),
            pl.BlockSpec((D, D * D), lambda i: (0, 0)),
            pl.BlockSpec((D * D, D), lambda i: (0, 0)),
        ],
        out_specs=pl.BlockSpec((eb, 4 * D), lambda i: (i, 0)),
        out_shape=jax.ShapeDtypeStruct((N_EDGES, 4 * D), jnp.float32),
    )(eh, csw, sq, dq, w2t, b2p, tmat, gmat)


def _upd_body(a0_ref, a1_ref, c0_ref, c1_ref, s_ref, root_ref, cb_ref,
              wri_ref, wzi_ref, wni_ref, wrh_ref, wzh_ref, wnh_ref,
              bi_ref, bh_ref, out_ref):
    cnt = jnp.maximum(c0_ref[...][:, :1] + c1_ref[...][:, :1], 1.0)
    agg = (a0_ref[...] + a1_ref[...]) / cnt
    s = s_ref[...]
    m = agg + jnp.dot(s, root_ref[...], preferred_element_type=jnp.float32)
    m = jnp.maximum(m + cb_ref[...], 0.0)
    bi = bi_ref[...]
    bh = bh_ref[...]
    gir = jnp.dot(m, wri_ref[...], preferred_element_type=jnp.float32) + bi[:, :D]
    giz = jnp.dot(m, wzi_ref[...], preferred_element_type=jnp.float32) + bi[:, D:2 * D]
    gin = jnp.dot(m, wni_ref[...], preferred_element_type=jnp.float32) + bi[:, 2 * D:]
    ghr = jnp.dot(s, wrh_ref[...], preferred_element_type=jnp.float32) + bh[:, :D]
    ghz = jnp.dot(s, wzh_ref[...], preferred_element_type=jnp.float32) + bh[:, D:2 * D]
    ghn = jnp.dot(s, wnh_ref[...], preferred_element_type=jnp.float32) + bh[:, 2 * D:]
    r = jax.nn.sigmoid(gir + ghr)
    z = jax.nn.sigmoid(giz + ghz)
    n = jnp.tanh(gin + r * ghn)
    out_ref[...] = (1.0 - z) * n + z * s


def _update(aggp, cntp, s, root, cb, wsplits, bi, bh):
    nb = 4096
    specs = [pl.BlockSpec((nb, D), lambda i: (i, 0)) for _ in range(5)]
    wspec = [pl.BlockSpec((D, D), lambda i: (0, 0)) for _ in range(7)]
    bspec = [pl.BlockSpec((1, 3 * D), lambda i: (0, 0)) for _ in range(2)]
    cbspec = [pl.BlockSpec((1, D), lambda i: (0, 0))]
    return pl.pallas_call(
        _upd_body,
        grid=(N_NODES // nb,),
        in_specs=specs + wspec[:1] + cbspec + wspec[1:] + bspec,
        out_specs=pl.BlockSpec((nb, D), lambda i: (i, 0)),
        out_shape=jax.ShapeDtypeStruct((N_NODES, D), jnp.float32),
    )(aggp[0], aggp[1], cntp[0], cntp[1], s, root, cb, *wsplits, bi, bh)


def _head_body(s_ref, w1_ref, b1_ref, w2_ref, b2_ref, w3_ref, b3_ref,
               sel_ref, mu_ref, std_ref):
    t1 = jnp.dot(s_ref[...], w1_ref[...], preferred_element_type=jnp.float32)
    t1 = jnp.maximum(t1 + b1_ref[...], 0.0)
    t2 = jnp.dot(t1, w2_ref[...], preferred_element_type=jnp.float32)
    t2 = jnp.maximum(t2 + b2_ref[...], 0.0)
    mix = jnp.dot(t2, w3_ref[...], preferred_element_type=jnp.float32) + b3_ref[...]
    sel = sel_ref[...]
    lane = lax.broadcasted_iota(jnp.int32, mix.shape, 1)
    oh = (lane == sel).astype(jnp.float32)
    mu_ref[...] = jnp.sum(mix * oh, axis=1, keepdims=True)
    mm = jnp.mean(mix, axis=1, keepdims=True)
    var = jnp.sum((mix - mm) * (mix - mm), axis=1, keepdims=True) / (MIX_N - 1.0)
    std_ref[...] = jnp.sqrt(var + VAR_EPS)


def _heads(s, w1, b1, w2, b2, w3, b3, sel):
    nb = 4096
    kd = MIX_N * D
    return pl.pallas_call(
        _head_body,
        grid=(N_NODES // nb,),
        in_specs=[
            pl.BlockSpec((nb, D), lambda i: (i, 0)),
            pl.BlockSpec((D, kd), lambda i: (0, 0)),
            pl.BlockSpec((1, kd), lambda i: (0, 0)),
            pl.BlockSpec((kd, kd), lambda i: (0, 0)),
            pl.BlockSpec((1, kd), lambda i: (0, 0)),
            pl.BlockSpec((kd, MIX_N), lambda i: (0, 0)),
            pl.BlockSpec((1, MIX_N), lambda i: (0, 0)),
            pl.BlockSpec((nb, 1), lambda i: (i, 0)),
        ],
        out_specs=[
            pl.BlockSpec((nb, 1), lambda i: (i, 0)),
            pl.BlockSpec((nb, 1), lambda i: (i, 0)),
        ],
        out_shape=[
            jax.ShapeDtypeStruct((N_NODES, 1), jnp.float32),
            jax.ShapeDtypeStruct((N_NODES, 1), jnp.float32),
        ],
    )(s, w1, b1, w2, b2, w3, b3, sel)


def _gather_body(cur_hbm, src_hbm, out_hbm, idx_v, rows_v, sem):
    c = lax.axis_index("c")
    s = lax.axis_index("s")
    wid = s * 2 + c
    pltpu.sync_copy(src_hbm.at[pl.ds(wid * NCH, NCH)], idx_v)
    for t in range(4):
        copies = []
        for j in range(4):
            copies.append(pltpu.async_copy(
                cur_hbm.at[idx_v.at[4 * t + j]],
                rows_v.at[pl.ds(j * CHUNK, CHUNK)], sem))
        for cp in copies:
            cp.wait()
        pltpu.sync_copy(rows_v, out_hbm.at[pl.ds(wid * EPW + t * 512, 512)])


def _sc_gather(cur4, srcq2d):
    mesh = plsc.VectorSubcoreMesh(core_axis_name="c", subcore_axis_name="s")
    return pl.kernel(
        _gather_body,
        out_type=jax.ShapeDtypeStruct((N_EDGES, 4 * D), jnp.float32),
        mesh=mesh,
        scratch_types=[
            pltpu.VMEM((NCH, CHUNK), jnp.int32),
            pltpu.VMEM((512, 4 * D), jnp.float32),
            pltpu.SemaphoreType.DMA,
        ],
    )(cur4, srcq2d)


def _scatter_body(msg_hbm, dst_hbm, zero_hbm, out_hbm, idx_v, rows_v, acc_sh, sem):
    c = lax.axis_index("c")
    s = lax.axis_index("s")
    wid = s * 2 + c
    rps = (N_NODES // 4) // 16
    pltpu.sync_copy(zero_hbm, acc_sh.at[pl.ds(s * rps, rps)])
    pltpu.sync_copy(dst_hbm.at[pl.ds(wid * NCH, NCH)], idx_v)
    plsc.subcore_barrier()
    for t in range(4):
        pltpu.sync_copy(msg_hbm.at[pl.ds(wid * EPW + t * 512, 512)], rows_v)
        for j in range(4):
            pltpu.sync_copy(rows_v.at[pl.ds(j * CHUNK, CHUNK)],
                            acc_sh.at[idx_v.at[4 * t + j]], add=True)
    plsc.subcore_barrier()
    pltpu.sync_copy(acc_sh.at[pl.ds(s * rps, rps)],
                    out_hbm.at[c, pl.ds(s * rps, rps)])


def _sc_scatter(msg, dstq2d, zero_rows):
    mesh = plsc.VectorSubcoreMesh(core_axis_name="c", subcore_axis_name="s")
    return pl.kernel(
        _scatter_body,
        out_type=jax.ShapeDtypeStruct((2, N_NODES // 4, 4 * D), jnp.float32),
        mesh=mesh,
        scratch_types=[
            pltpu.VMEM((NCH, CHUNK), jnp.int32),
            pltpu.VMEM((512, 4 * D), jnp.float32),
            pltpu.VMEM_SHARED((N_NODES // 4, 4 * D), jnp.float32),
            pltpu.SemaphoreType.DMA,
        ],
    )(msg, dstq2d, zero_rows)


def kernel(x, edge_index, edge_attr, input_idx, W_embed, b_embed, bn_gamma,
           bn_beta, We1, be1, We2, be2, root, conv_bias, Wih, Whh, bih, bhh,
           mW1, mb1, mW2, mb2, mW3, mb3):
    f32 = jnp.float32
    src = edge_index[0]
    dst = edge_index[1]
    srcq2d = (src // 4).reshape(N_EDGES // CHUNK, CHUNK)
    dstq2d = (dst // 4).reshape(N_EDGES // CHUNK, CHUNK)
    sq = jnp.mod(src, 4).reshape(N_EDGES, 1)
    dq = jnp.mod(dst, 4).reshape(N_EDGES, 1)

    wt = W_embed.T
    b2 = b_embed.reshape(1, D)
    g2 = bn_gamma.reshape(1, D)
    bt2 = bn_beta.reshape(1, D)
    w1t = We1.T
    be1r = be1.reshape(1, EL)
    we2p = We2.reshape(D, D, EL).transpose(1, 0, 2).reshape(D * D, EL)
    w2t = we2p.T.astype(jnp.bfloat16)
    be2p = be2.reshape(D, D).T.reshape(1, D * D)
    eye = jnp.eye(D, dtype=f32)
    tmat = jnp.tile(eye, (1, D))
    gmat = jnp.repeat(eye, D, axis=0)
    cb = conv_bias.reshape(1, D)
    wsplits = (Wih[:D].T, Wih[D:2 * D].T, Wih[2 * D:].T,
               Whh[:D].T, Whh[D:2 * D].T, Whh[2 * D:].T)
    bi = bih.reshape(1, 3 * D)
    bh = bhh.reshape(1, 3 * D)
    kd = MIX_N * D
    w1 = jnp.transpose(mW1, (2, 0, 1)).reshape(D, kd)
    b1 = mb1.reshape(1, kd)
    w2bd = jax.scipy.linalg.block_diag(*[mW2[k].T for k in range(MIX_N)])
    b2h = mb2.reshape(1, kd)
    w3bd = jax.scipy.linalg.block_diag(*[mW3[k].T for k in range(MIX_N)])
    b3h = mb3.reshape(1, MIX_N)
    sel = jnp.repeat(jnp.mod(input_idx, MIX_N), MAX_N).reshape(N_NODES, 1)
    sel = sel.astype(jnp.int32)
    zero_rows = jnp.zeros(((N_NODES // 4) // 16, 4 * D), f32)
    onesw = (lax.broadcasted_iota(jnp.int32, (N_EDGES, 4), 1) == dq).astype(f32)
    onesw = jnp.repeat(onesw, D, axis=1)

    y = _embed(x, wt, b2, g2, bt2)
    eh = _edges(edge_attr, w1t, be1r)
    cntp = _sc_scatter(onesw, dstq2d, zero_rows).reshape(2, N_NODES, D)

    s = y
    for _ in range(3):
        cs = _sc_gather(s.reshape(N_NODES // 4, 4 * D), srcq2d)
        msg = _msg(eh, cs, sq, dq, w2t, be2p, tmat, gmat)
        aggp = _sc_scatter(msg, dstq2d, zero_rows).reshape(2, N_NODES, D)
        s = _update(aggp, cntp, s, root, cb, wsplits, bi, bh)

    mu, std = _heads(s, w1, b1, w2bd, b2h, w3bd, b3h, sel)
    return (mu.reshape(N_NODES // MAX_N, MAX_N, 1),
            std.reshape(N_NODES // MAX_N, MAX_N, 1))


# speedup vs baseline: 2.3152x; 1.2060x over previous
import functools

import jax
import jax.numpy as jnp
from jax import lax
from jax.experimental import pallas as pl
from jax.experimental.pallas import tpu as pltpu
from jax.experimental.pallas import tpu_sc as plsc

N_NODES = 16384
N_EDGES = 65536
D_IN = 128
D = 32
EL = 128
MIX_N = 10
MAX_N = 64
BN_EPS = 1e-5
VAR_EPS = 1e-5

NW = 32
EPW = N_EDGES // NW
CHUNK = 128
NCH = EPW // CHUNK


def _embed_body(x_ref, wt_ref, b_ref, g_ref, bt_ref, y_ref):
    y = jnp.dot(x_ref[...], wt_ref[...], preferred_element_type=jnp.float32)
    y = y + b_ref[...]
    mean = jnp.mean(y, axis=0, keepdims=True)
    var = jnp.mean((y - mean) * (y - mean), axis=0, keepdims=True)
    y_ref[...] = (y - mean) * lax.rsqrt(var + BN_EPS) * g_ref[...] + bt_ref[...]


def _embed(x, wt, b, g, bt):
    return pl.pallas_call(
        _embed_body,
        out_shape=jax.ShapeDtypeStruct((N_NODES, D), jnp.float32),
    )(x, wt, b, g, bt)


def _edges_body(ea_ref, w1t_ref, b1_ref, eh_ref):
    t = jnp.dot(ea_ref[...], w1t_ref[...], preferred_element_type=jnp.float32)
    eh_ref[...] = jax.nn.sigmoid(t + b1_ref[...]).astype(jnp.bfloat16)


def _edges(ea, w1t, b1):
    eb = 8192
    return pl.pallas_call(
        _edges_body,
        grid=(N_EDGES // eb,),
        in_specs=[
            pl.BlockSpec((eb, 4), lambda i: (i, 0)),
            pl.BlockSpec((4, EL), lambda i: (0, 0)),
            pl.BlockSpec((1, EL), lambda i: (0, 0)),
        ],
        out_specs=pl.BlockSpec((eb, EL), lambda i: (i, 0)),
        out_shape=jax.ShapeDtypeStruct((N_EDGES, EL), jnp.bfloat16),
    )(ea, w1t, b1)


def _msg_body(eh_ref, cs_ref, w2_ref, b2_ref, t_ref, g_ref, msg_ref):
    f32 = jnp.float32
    wg = jnp.dot(eh_ref[...], w2_ref[...], preferred_element_type=f32)
    wg = wg + b2_ref[...]
    ct = jnp.dot(cs_ref[...], t_ref[...], preferred_element_type=f32)
    msg_ref[...] = jnp.dot(wg * ct, g_ref[...], preferred_element_type=f32)


def _msg(eh, cs, w2t, b2p, tmat, gmat):
    eb = 1024
    return pl.pallas_call(
        _msg_body,
        grid=(N_EDGES // eb,),
        in_specs=[
            pl.BlockSpec((eb, EL), lambda i: (i, 0)),
            pl.BlockSpec((eb, D), lambda i: (i, 0)),
            pl.BlockSpec((EL, D * D), lambda i: (0, 0)),
            pl.BlockSpec((1, D * D), lambda i: (0, 0)),
            pl.BlockSpec((D, D * D), lambda i: (0, 0)),
            pl.BlockSpec((D * D, D), lambda i: (0, 0)),
        ],
        out_specs=pl.BlockSpec((eb, D), lambda i: (i, 0)),
        out_shape=jax.ShapeDtypeStruct((N_EDGES, D), jnp.float32),
    )(eh, cs, w2t, b2p, tmat, gmat)


def _upd_body(a0_ref, a1_ref, c0_ref, c1_ref, s_ref, root_ref, cb_ref,
              wri_ref, wzi_ref, wni_ref, wrh_ref, wzh_ref, wnh_ref,
              bi_ref, bh_ref, out_ref):
    cnt = jnp.maximum(c0_ref[...][:, :1] + c1_ref[...][:, :1], 1.0)
    agg = (a0_ref[...] + a1_ref[...]) / cnt
    s = s_ref[...]
    m = agg + jnp.dot(s, root_ref[...], preferred_element_type=jnp.float32)
    m = jnp.maximum(m + cb_ref[...], 0.0)
    bi = bi_ref[...]
    bh = bh_ref[...]
    gir = jnp.dot(m, wri_ref[...], preferred_element_type=jnp.float32) + bi[:, :D]
    giz = jnp.dot(m, wzi_ref[...], preferred_element_type=jnp.float32) + bi[:, D:2 * D]
    gin = jnp.dot(m, wni_ref[...], preferred_element_type=jnp.float32) + bi[:, 2 * D:]
    ghr = jnp.dot(s, wrh_ref[...], preferred_element_type=jnp.float32) + bh[:, :D]
    ghz = jnp.dot(s, wzh_ref[...], preferred_element_type=jnp.float32) + bh[:, D:2 * D]
    ghn = jnp.dot(s, wnh_ref[...], preferred_element_type=jnp.float32) + bh[:, 2 * D:]
    r = jax.nn.sigmoid(gir + ghr)
    z = jax.nn.sigmoid(giz + ghz)
    n = jnp.tanh(gin + r * ghn)
    out_ref[...] = (1.0 - z) * n + z * s


def _update(aggp, cntp, s, root, cb, wsplits, bi, bh):
    nb = 4096
    specs = [pl.BlockSpec((nb, D), lambda i: (i, 0)) for _ in range(5)]
    wspec = [pl.BlockSpec((D, D), lambda i: (0, 0)) for _ in range(7)]
    bspec = [pl.BlockSpec((1, 3 * D), lambda i: (0, 0)) for _ in range(2)]
    cbspec = [pl.BlockSpec((1, D), lambda i: (0, 0))]
    return pl.pallas_call(
        _upd_body,
        grid=(N_NODES // nb,),
        in_specs=specs + wspec[:1] + cbspec + wspec[1:] + bspec,
        out_specs=pl.BlockSpec((nb, D), lambda i: (i, 0)),
        out_shape=jax.ShapeDtypeStruct((N_NODES, D), jnp.float32),
    )(aggp[0], aggp[1], cntp[0], cntp[1], s, root, cb, *wsplits, bi, bh)


def _head_body(s_ref, w1_ref, b1_ref, w2_ref, b2_ref, w3_ref, b3_ref,
               sel_ref, mu_ref, std_ref):
    t1 = jnp.dot(s_ref[...], w1_ref[...], preferred_element_type=jnp.float32)
    t1 = jnp.maximum(t1 + b1_ref[...], 0.0)
    t2 = jnp.dot(t1, w2_ref[...], preferred_element_type=jnp.float32)
    t2 = jnp.maximum(t2 + b2_ref[...], 0.0)
    mix = jnp.dot(t2, w3_ref[...], preferred_element_type=jnp.float32) + b3_ref[...]
    sel = sel_ref[...]
    lane = lax.broadcasted_iota(jnp.int32, mix.shape, 1)
    oh = (lane == sel).astype(jnp.float32)
    mu_ref[...] = jnp.sum(mix * oh, axis=1, keepdims=True)
    mm = jnp.mean(mix, axis=1, keepdims=True)
    var = jnp.sum((mix - mm) * (mix - mm), axis=1, keepdims=True) / (MIX_N - 1.0)
    std_ref[...] = jnp.sqrt(var + VAR_EPS)


def _heads(s, w1, b1, w2, b2, w3, b3, sel):
    nb = 4096
    kd = MIX_N * D
    return pl.pallas_call(
        _head_body,
        grid=(N_NODES // nb,),
        in_specs=[
            pl.BlockSpec((nb, D), lambda i: (i, 0)),
            pl.BlockSpec((D, kd), lambda i: (0, 0)),
            pl.BlockSpec((1, kd), lambda i: (0, 0)),
            pl.BlockSpec((kd, kd), lambda i: (0, 0)),
            pl.BlockSpec((1, kd), lambda i: (0, 0)),
            pl.BlockSpec((kd, MIX_N), lambda i: (0, 0)),
            pl.BlockSpec((1, MIX_N), lambda i: (0, 0)),
            pl.BlockSpec((nb, 1), lambda i: (i, 0)),
        ],
        out_specs=[
            pl.BlockSpec((nb, 1), lambda i: (i, 0)),
            pl.BlockSpec((nb, 1), lambda i: (i, 0)),
        ],
        out_shape=[
            jax.ShapeDtypeStruct((N_NODES, 1), jnp.float32),
            jax.ShapeDtypeStruct((N_NODES, 1), jnp.float32),
        ],
    )(s, w1, b1, w2, b2, w3, b3, sel)


def _gather_body(cur_hbm, src_hbm, out_hbm, idx_v, rows_v, tab_sh, sem):
    c = lax.axis_index("c")
    s = lax.axis_index("s")
    wid = s * 2 + c
    nps = N_NODES // 16
    pltpu.sync_copy(cur_hbm.at[pl.ds(s * nps, nps)], tab_sh.at[pl.ds(s * nps, nps)])
    pltpu.sync_copy(src_hbm.at[pl.ds(wid * NCH, NCH)], idx_v)
    plsc.subcore_barrier()
    copies = []
    for j in range(NCH):
        copies.append(pltpu.async_copy(
            tab_sh.at[idx_v.at[j]], rows_v.at[pl.ds(j * CHUNK, CHUNK)], sem))
    for cp in copies:
        cp.wait()
    pltpu.sync_copy(rows_v, out_hbm.at[pl.ds(wid * EPW, EPW)])


def _sc_gather(cur, src2d):
    mesh = plsc.VectorSubcoreMesh(core_axis_name="c", subcore_axis_name="s")
    return pl.kernel(
        _gather_body,
        out_type=jax.ShapeDtypeStruct((N_EDGES, D), jnp.float32),
        mesh=mesh,
        compiler_params=pltpu.CompilerParams(use_tc_tiling_on_sc=False),
        scratch_types=[
            pltpu.VMEM((NCH, CHUNK), jnp.int32),
            pltpu.VMEM((EPW, D), jnp.float32),
            pltpu.VMEM_SHARED((N_NODES, D), jnp.float32),
            pltpu.SemaphoreType.DMA,
        ],
    )(cur, src2d)


def _scatter_body(msg_hbm, dst_hbm, zero_hbm, out_hbm, idx_v, rows_v, acc_sh, sem):
    c = lax.axis_index("c")
    s = lax.axis_index("s")
    wid = s * 2 + c
    rps = N_NODES // 16
    pltpu.sync_copy(zero_hbm, acc_sh.at[pl.ds(s * rps, rps)])
    pltpu.sync_copy(dst_hbm.at[pl.ds(wid * NCH, NCH)], idx_v)
    pltpu.sync_copy(msg_hbm.at[pl.ds(wid * EPW, EPW)], rows_v)
    plsc.subcore_barrier()
    for j in range(NCH):
        pltpu.sync_copy(rows_v.at[pl.ds(j * CHUNK, CHUNK)],
                        acc_sh.at[idx_v.at[j]], add=True)
    plsc.subcore_barrier()
    pltpu.sync_copy(acc_sh.at[pl.ds(s * rps, rps)],
                    out_hbm.at[c, pl.ds(s * rps, rps)])


def _sc_scatter(msg, dst2d, zero_rows):
    mesh = plsc.VectorSubcoreMesh(core_axis_name="c", subcore_axis_name="s")
    return pl.kernel(
        _scatter_body,
        out_type=jax.ShapeDtypeStruct((2, N_NODES, D), jnp.float32),
        mesh=mesh,
        compiler_params=pltpu.CompilerParams(use_tc_tiling_on_sc=False),
        scratch_types=[
            pltpu.VMEM((NCH, CHUNK), jnp.int32),
            pltpu.VMEM((EPW, D), jnp.float32),
            pltpu.VMEM_SHARED((N_NODES, D), jnp.float32),
            pltpu.SemaphoreType.DMA,
        ],
    )(msg, dst2d, zero_rows)


def kernel(x, edge_index, edge_attr, input_idx, W_embed, b_embed, bn_gamma,
           bn_beta, We1, be1, We2, be2, root, conv_bias, Wih, Whh, bih, bhh,
           mW1, mb1, mW2, mb2, mW3, mb3):
    f32 = jnp.float32
    src2d = edge_index[0].reshape(N_EDGES // CHUNK, CHUNK)
    dst2d = edge_index[1].reshape(N_EDGES // CHUNK, CHUNK)

    wt = W_embed.T
    b2 = b_embed.reshape(1, D)
    g2 = bn_gamma.reshape(1, D)
    bt2 = bn_beta.reshape(1, D)
    w1t = We1.T
    be1r = be1.reshape(1, EL)
    we2p = We2.reshape(D, D, EL).transpose(1, 0, 2).reshape(D * D, EL)
    w2t = we2p.T.astype(jnp.bfloat16)
    be2p = be2.reshape(D, D).T.reshape(1, D * D)
    eye = jnp.eye(D, dtype=f32)
    tmat = jnp.tile(eye, (1, D))
    gmat = jnp.repeat(eye, D, axis=0)
    cb = conv_bias.reshape(1, D)
    wsplits = (Wih[:D].T, Wih[D:2 * D].T, Wih[2 * D:].T,
               Whh[:D].T, Whh[D:2 * D].T, Whh[2 * D:].T)
    bi = bih.reshape(1, 3 * D)
    bh = bhh.reshape(1, 3 * D)
    kd = MIX_N * D
    w1 = jnp.transpose(mW1, (2, 0, 1)).reshape(D, kd)
    b1 = mb1.reshape(1, kd)
    w2bd = jax.scipy.linalg.block_diag(*[mW2[k].T for k in range(MIX_N)])
    b2h = mb2.reshape(1, kd)
    w3bd = jax.scipy.linalg.block_diag(*[mW3[k].T for k in range(MIX_N)])
    b3h = mb3.reshape(1, MIX_N)
    sel = jnp.repeat(jnp.mod(input_idx, MIX_N), MAX_N).reshape(N_NODES, 1)
    sel = sel.astype(jnp.int32)
    zero_rows = jnp.zeros((N_NODES // 16, D), f32)
    ones_rows = jnp.ones((N_EDGES, D), f32)

    y = _embed(x, wt, b2, g2, bt2)
    eh = _edges(edge_attr, w1t, be1r)
    cntp = _sc_scatter(ones_rows, dst2d, zero_rows)

    s = y
    for _ in range(3):
        cs = _sc_gather(s, src2d)
        msg = _msg(eh, cs, w2t, be2p, tmat, gmat)
        aggp = _sc_scatter(msg, dst2d, zero_rows)
        s = _update(aggp, cntp, s, root, cb, wsplits, bi, bh)

    mu, std = _heads(s, w1, b1, w2bd, b2h, w3bd, b3h, sel)
    return (mu.reshape(N_NODES // MAX_N, MAX_N, 1),
            std.reshape(N_NODES // MAX_N, MAX_N, 1))


# speedup vs baseline: 2.3827x; 1.0292x over previous
import functools

import jax
import jax.numpy as jnp
from jax import lax
from jax.experimental import pallas as pl
from jax.experimental.pallas import tpu as pltpu
from jax.experimental.pallas import tpu_sc as plsc

N_NODES = 16384
N_EDGES = 65536
D_IN = 128
D = 32
EL = 128
MIX_N = 10
MAX_N = 64
BN_EPS = 1e-5
VAR_EPS = 1e-5

NW = 32
EPW = N_EDGES // NW
CHUNK = 128
NCH = EPW // CHUNK


def _pre_body(x_ref, wt_ref, b_ref, g_ref, bt_ref, ea_ref, w1t_ref, b1_ref,
              y_ref, eh_ref):
    i = pl.program_id(0)

    @pl.when(i == 0)
    def _():
        y = jnp.dot(x_ref[...], wt_ref[...],
                    preferred_element_type=jnp.float32)
        y = y + b_ref[...]
        mean = jnp.mean(y, axis=0, keepdims=True)
        var = jnp.mean((y - mean) * (y - mean), axis=0, keepdims=True)
        y_ref[...] = ((y - mean) * lax.rsqrt(var + BN_EPS) * g_ref[...]
                      + bt_ref[...])

    t = jnp.dot(ea_ref[...], w1t_ref[...], preferred_element_type=jnp.float32)
    eh_ref[...] = jax.nn.sigmoid(t + b1_ref[...]).astype(jnp.bfloat16)


def _pre(x, wt, b, g, bt, ea, w1t, b1):
    eb = 8192
    return pl.pallas_call(
        _pre_body,
        grid=(N_EDGES // eb,),
        in_specs=[
            pl.BlockSpec((N_NODES, D_IN), lambda i: (0, 0)),
            pl.BlockSpec((D_IN, D), lambda i: (0, 0)),
            pl.BlockSpec((1, D), lambda i: (0, 0)),
            pl.BlockSpec((1, D), lambda i: (0, 0)),
            pl.BlockSpec((1, D), lambda i: (0, 0)),
            pl.BlockSpec((eb, 4), lambda i: (i, 0)),
            pl.BlockSpec((4, EL), lambda i: (0, 0)),
            pl.BlockSpec((1, EL), lambda i: (0, 0)),
        ],
        out_specs=[
            pl.BlockSpec((N_NODES, D), lambda i: (0, 0)),
            pl.BlockSpec((eb, EL), lambda i: (i, 0)),
        ],
        out_shape=[
            jax.ShapeDtypeStruct((N_NODES, D), jnp.float32),
            jax.ShapeDtypeStruct((N_EDGES, EL), jnp.bfloat16),
        ],
    )(x, wt, b, g, bt, ea, w1t, b1)


def _msg_body(eh_ref, cs_ref, w2_ref, b2_ref, t_ref, g_ref, msg_ref):
    f32 = jnp.float32
    bf16 = jnp.bfloat16
    wg = jnp.dot(eh_ref[...], w2_ref[...], preferred_element_type=f32)
    wg = (wg + b2_ref[...]).astype(bf16)
    ct = jnp.dot(cs_ref[...].astype(bf16), t_ref[...],
                 preferred_element_type=f32).astype(bf16)
    msg_ref[...] = jnp.dot(wg * ct, g_ref[...], preferred_element_type=f32)


def _msg(eh, cs, w2t, b2p, tmat, gmat):
    eb = 2048
    return pl.pallas_call(
        _msg_body,
        grid=(N_EDGES // eb,),
        in_specs=[
            pl.BlockSpec((eb, EL), lambda i: (i, 0)),
            pl.BlockSpec((eb, D), lambda i: (i, 0)),
            pl.BlockSpec((EL, D * D), lambda i: (0, 0)),
            pl.BlockSpec((1, D * D), lambda i: (0, 0)),
            pl.BlockSpec((D, D * D), lambda i: (0, 0)),
            pl.BlockSpec((D * D, D), lambda i: (0, 0)),
        ],
        out_specs=pl.BlockSpec((eb, D), lambda i: (i, 0)),
        out_shape=jax.ShapeDtypeStruct((N_EDGES, D), jnp.float32),
    )(eh, cs, w2t, b2p, tmat, gmat)


def _gru_compute(a0_ref, a1_ref, c0_ref, c1_ref, s_ref, root_ref, cb_ref,
                 wri_ref, wzi_ref, wni_ref, wrh_ref, wzh_ref, wnh_ref,
                 bi_ref, bh_ref):
    f32 = jnp.float32
    cnt = jnp.maximum(c0_ref[...][:, :1] + c1_ref[...][:, :1], 1.0)
    agg = (a0_ref[...] + a1_ref[...]) / cnt
    s = s_ref[...]
    m = agg + jnp.dot(s, root_ref[...], preferred_element_type=f32)
    m = jnp.maximum(m + cb_ref[...], 0.0)
    bi = bi_ref[...]
    bh = bh_ref[...]
    gir = jnp.dot(m, wri_ref[...], preferred_element_type=f32) + bi[:, :D]
    giz = jnp.dot(m, wzi_ref[...], preferred_element_type=f32) + bi[:, D:2 * D]
    gin = jnp.dot(m, wni_ref[...], preferred_element_type=f32) + bi[:, 2 * D:]
    ghr = jnp.dot(s, wrh_ref[...], preferred_element_type=f32) + bh[:, :D]
    ghz = jnp.dot(s, wzh_ref[...], preferred_element_type=f32) + bh[:, D:2 * D]
    ghn = jnp.dot(s, wnh_ref[...], preferred_element_type=f32) + bh[:, 2 * D:]
    r = jax.nn.sigmoid(gir + ghr)
    z = jax.nn.sigmoid(giz + ghz)
    n = jnp.tanh(gin + r * ghn)
    return (1.0 - z) * n + z * s


def _upd_body(*refs):
    out_ref = refs[-1]
    out_ref[...] = _gru_compute(*refs[:-1])


_NODE_SPECS = None


def _gru_specs(nb):
    specs = [pl.BlockSpec((nb, D), lambda i: (i, 0)) for _ in range(5)]
    wspec = [pl.BlockSpec((D, D), lambda i: (0, 0)) for _ in range(7)]
    bspec = [pl.BlockSpec((1, 3 * D), lambda i: (0, 0)) for _ in range(2)]
    cbspec = [pl.BlockSpec((1, D), lambda i: (0, 0))]
    return specs + wspec[:1] + cbspec + wspec[1:] + bspec


def _update(aggp, cntp, s, root, cb, wsplits, bi, bh):
    nb = 4096
    return pl.pallas_call(
        _upd_body,
        grid=(N_NODES // nb,),
        in_specs=_gru_specs(nb),
        out_specs=pl.BlockSpec((nb, D), lambda i: (i, 0)),
        out_shape=jax.ShapeDtypeStruct((N_NODES, D), jnp.float32),
    )(aggp[0], aggp[1], cntp[0], cntp[1], s, root, cb, *wsplits, bi, bh)


def _updhead_body(*refs):
    mu_ref, std_ref = refs[-2], refs[-1]
    gru_refs = refs[:15]
    w1_ref, b1_ref, w2_ref, b2_ref, w3_ref, b3_ref, sel_ref = refs[15:22]
    h = _gru_compute(*gru_refs)
    t1 = jnp.dot(h, w1_ref[...], preferred_element_type=jnp.float32)
    t1 = jnp.maximum(t1 + b1_ref[...], 0.0)
    t2 = jnp.dot(t1, w2_ref[...], preferred_element_type=jnp.float32)
    t2 = jnp.maximum(t2 + b2_ref[...], 0.0)
    mix = (jnp.dot(t2, w3_ref[...], preferred_element_type=jnp.float32)
           + b3_ref[...])
    sel = sel_ref[...]
    lane = lax.broadcasted_iota(jnp.int32, mix.shape, 1)
    oh = (lane == sel).astype(jnp.float32)
    mu_ref[...] = jnp.sum(mix * oh, axis=1, keepdims=True)
    mm = jnp.mean(mix, axis=1, keepdims=True)
    var = jnp.sum((mix - mm) * (mix - mm), axis=1, keepdims=True) / (MIX_N - 1.0)
    std_ref[...] = jnp.sqrt(var + VAR_EPS)


def _updheads(aggp, cntp, s, root, cb, wsplits, bi, bh,
              w1, b1, w2, b2, w3, b3, sel):
    nb = 4096
    kd = MIX_N * D
    head_specs = [
        pl.BlockSpec((D, kd), lambda i: (0, 0)),
        pl.BlockSpec((1, kd), lambda i: (0, 0)),
        pl.BlockSpec((kd, kd), lambda i: (0, 0)),
        pl.BlockSpec((1, kd), lambda i: (0, 0)),
        pl.BlockSpec((kd, MIX_N), lambda i: (0, 0)),
        pl.BlockSpec((1, MIX_N), lambda i: (0, 0)),
        pl.BlockSpec((nb, 1), lambda i: (i, 0)),
    ]
    return pl.pallas_call(
        _updhead_body,
        grid=(N_NODES // nb,),
        in_specs=_gru_specs(nb) + head_specs,
        out_specs=[
            pl.BlockSpec((nb, 1), lambda i: (i, 0)),
            pl.BlockSpec((nb, 1), lambda i: (i, 0)),
        ],
        out_shape=[
            jax.ShapeDtypeStruct((N_NODES, 1), jnp.float32),
            jax.ShapeDtypeStruct((N_NODES, 1), jnp.float32),
        ],
    )(aggp[0], aggp[1], cntp[0], cntp[1], s, root, cb, *wsplits, bi, bh,
      w1, b1, w2, b2, w3, b3, sel)


def _gather_body(cur_hbm, src_hbm, out_hbm, idx_v, rows_v, tab_sh, sem):
    c = lax.axis_index("c")
    s = lax.axis_index("s")
    wid = s * 2 + c
    nps = N_NODES // 16
    pltpu.sync_copy(cur_hbm.at[pl.ds(s * nps, nps)], tab_sh.at[pl.ds(s * nps, nps)])
    pltpu.sync_copy(src_hbm.at[pl.ds(wid * NCH, NCH)], idx_v)
    plsc.subcore_barrier()
    copies = []
    for j in range(NCH):
        copies.append(pltpu.async_copy(
            tab_sh.at[idx_v.at[j]], rows_v.at[pl.ds(j * CHUNK, CHUNK)], sem))
    for cp in copies:
        cp.wait()
    pltpu.sync_copy(rows_v, out_hbm.at[pl.ds(wid * EPW, EPW)])


def _sc_gather(cur, src2d):
    mesh = plsc.VectorSubcoreMesh(core_axis_name="c", subcore_axis_name="s")
    return pl.kernel(
        _gather_body,
        out_type=jax.ShapeDtypeStruct((N_EDGES, D), jnp.float32),
        mesh=mesh,
        compiler_params=pltpu.CompilerParams(use_tc_tiling_on_sc=False),
        scratch_types=[
            pltpu.VMEM((NCH, CHUNK), jnp.int32),
            pltpu.VMEM((EPW, D), jnp.float32),
            pltpu.VMEM_SHARED((N_NODES, D), jnp.float32),
            pltpu.SemaphoreType.DMA,
        ],
    )(cur, src2d)


def _scatter_body(msg_hbm, dst_hbm, zero_hbm, out_hbm, idx_v, rows_v, acc_sh, sem):
    c = lax.axis_index("c")
    s = lax.axis_index("s")
    wid = s * 2 + c
    rps = N_NODES // 16
    pltpu.sync_copy(zero_hbm, acc_sh.at[pl.ds(s * rps, rps)])
    pltpu.sync_copy(dst_hbm.at[pl.ds(wid * NCH, NCH)], idx_v)
    pltpu.sync_copy(msg_hbm.at[pl.ds(wid * EPW, EPW)], rows_v)
    plsc.subcore_barrier()
    for j in range(NCH):
        pltpu.sync_copy(rows_v.at[pl.ds(j * CHUNK, CHUNK)],
                        acc_sh.at[idx_v.at[j]], add=True)
    plsc.subcore_barrier()
    pltpu.sync_copy(acc_sh.at[pl.ds(s * rps, rps)],
                    out_hbm.at[c, pl.ds(s * rps, rps)])


def _sc_scatter(msg, dst2d, zero_rows):
    mesh = plsc.VectorSubcoreMesh(core_axis_name="c", subcore_axis_name="s")
    return pl.kernel(
        _scatter_body,
        out_type=jax.ShapeDtypeStruct((2, N_NODES, D), jnp.float32),
        mesh=mesh,
        compiler_params=pltpu.CompilerParams(use_tc_tiling_on_sc=False),
        scratch_types=[
            pltpu.VMEM((NCH, CHUNK), jnp.int32),
            pltpu.VMEM((EPW, D), jnp.float32),
            pltpu.VMEM_SHARED((N_NODES, D), jnp.float32),
            pltpu.SemaphoreType.DMA,
        ],
    )(msg, dst2d, zero_rows)


def kernel(x, edge_index, edge_attr, input_idx, W_embed, b_embed, bn_gamma,
           bn_beta, We1, be1, We2, be2, root, conv_bias, Wih, Whh, bih, bhh,
           mW1, mb1, mW2, mb2, mW3, mb3):
    f32 = jnp.float32
    src2d = edge_index[0].reshape(N_EDGES // CHUNK, CHUNK)
    dst2d = edge_index[1].reshape(N_EDGES // CHUNK, CHUNK)

    wt = W_embed.T
    b2 = b_embed.reshape(1, D)
    g2 = bn_gamma.reshape(1, D)
    bt2 = bn_beta.reshape(1, D)
    w1t = We1.T
    be1r = be1.reshape(1, EL)
    we2p = We2.reshape(D, D, EL).transpose(1, 0, 2).reshape(D * D, EL)
    w2t = we2p.T.astype(jnp.bfloat16)
    be2p = be2.reshape(D, D).T.reshape(1, D * D)
    eye = jnp.eye(D, dtype=jnp.bfloat16)
    tmat = jnp.tile(eye, (1, D))
    gmat = jnp.repeat(eye, D, axis=0)
    cb = conv_bias.reshape(1, D)
    wsplits = (Wih[:D].T, Wih[D:2 * D].T, Wih[2 * D:].T,
               Whh[:D].T, Whh[D:2 * D].T, Whh[2 * D:].T)
    bi = bih.reshape(1, 3 * D)
    bh = bhh.reshape(1, 3 * D)
    kd = MIX_N * D
    w1 = jnp.transpose(mW1, (2, 0, 1)).reshape(D, kd)
    b1 = mb1.reshape(1, kd)
    w2bd = jax.scipy.linalg.block_diag(*[mW2[k].T for k in range(MIX_N)])
    b2h = mb2.reshape(1, kd)
    w3bd = jax.scipy.linalg.block_diag(*[mW3[k].T for k in range(MIX_N)])
    b3h = mb3.reshape(1, MIX_N)
    sel = jnp.repeat(jnp.mod(input_idx, MIX_N), MAX_N).reshape(N_NODES, 1)
    sel = sel.astype(jnp.int32)
    zero_rows = jnp.zeros((N_NODES // 16, D), f32)
    ones_rows = jnp.ones((N_EDGES, D), f32)

    y, eh = _pre(x, wt, b2, g2, bt2, edge_attr, w1t, be1r)
    cntp = _sc_scatter(ones_rows, dst2d, zero_rows)

    s = y
    for it in range(3):
        cs = _sc_gather(s, src2d)
        msg = _msg(eh, cs, w2t, be2p, tmat, gmat)
        aggp = _sc_scatter(msg, dst2d, zero_rows)
        if it < 2:
            s = _update(aggp, cntp, s, root, cb, wsplits, bi, bh)
    mu, std = _updheads(aggp, cntp, s, root, cb, wsplits, bi, bh,
                        w1, b1, w2bd, b2h, w3bd, b3h, sel)
    return (mu.reshape(N_NODES // MAX_N, MAX_N, 1),
            std.reshape(N_NODES // MAX_N, MAX_N, 1))


# speedup vs baseline: 2.6236x; 1.1011x over previous
import functools

import jax
import jax.numpy as jnp
from jax import lax
from jax.experimental import pallas as pl
from jax.experimental.pallas import tpu as pltpu
from jax.experimental.pallas import tpu_sc as plsc

N_NODES = 16384
N_EDGES = 65536
D_IN = 128
D = 32
EL = 128
MIX_N = 10
MAX_N = 64
BN_EPS = 1e-5
VAR_EPS = 1e-5

NW = 32
EPW = N_EDGES // NW
CHUNK = 128
NCH = EPW // CHUNK


def _pre_body(x_ref, wt_ref, b_ref, g_ref, bt_ref, ea_ref, w1t_ref, b1_ref,
              w2_ref, b2_ref, y_ref, wg_ref):
    i = pl.program_id(0)

    @pl.when(i == 0)
    def _():
        y = jnp.dot(x_ref[...], wt_ref[...],
                    preferred_element_type=jnp.float32)
        y = y + b_ref[...]
        mean = jnp.mean(y, axis=0, keepdims=True)
        var = jnp.mean((y - mean) * (y - mean), axis=0, keepdims=True)
        y_ref[...] = ((y - mean) * lax.rsqrt(var + BN_EPS) * g_ref[...]
                      + bt_ref[...])

    t = jnp.dot(ea_ref[...], w1t_ref[...], preferred_element_type=jnp.float32)
    eh = jax.nn.sigmoid(t + b1_ref[...]).astype(jnp.bfloat16)
    wg = jnp.dot(eh, w2_ref[...], preferred_element_type=jnp.float32)
    wg_ref[...] = (wg + b2_ref[...]).astype(jnp.bfloat16)


def _pre(x, wt, b, g, bt, ea, w1t, b1, w2t, b2p):
    eb = 2048
    return pl.pallas_call(
        _pre_body,
        grid=(N_EDGES // eb,),
        in_specs=[
            pl.BlockSpec((N_NODES, D_IN), lambda i: (0, 0)),
            pl.BlockSpec((D_IN, D), lambda i: (0, 0)),
            pl.BlockSpec((1, D), lambda i: (0, 0)),
            pl.BlockSpec((1, D), lambda i: (0, 0)),
            pl.BlockSpec((1, D), lambda i: (0, 0)),
            pl.BlockSpec((eb, 4), lambda i: (i, 0)),
            pl.BlockSpec((4, EL), lambda i: (0, 0)),
            pl.BlockSpec((1, EL), lambda i: (0, 0)),
            pl.BlockSpec((EL, D * D), lambda i: (0, 0)),
            pl.BlockSpec((1, D * D), lambda i: (0, 0)),
        ],
        out_specs=[
            pl.BlockSpec((N_NODES, D), lambda i: (0, 0)),
            pl.BlockSpec((eb, D * D), lambda i: (i, 0)),
        ],
        out_shape=[
            jax.ShapeDtypeStruct((N_NODES, D), jnp.float32),
            jax.ShapeDtypeStruct((N_EDGES, D * D), jnp.bfloat16),
        ],
    )(x, wt, b, g, bt, ea, w1t, b1, w2t, b2p)


def _msg_body(wg_ref, cs_ref, t_ref, g_ref, msg_ref):
    f32 = jnp.float32
    bf16 = jnp.bfloat16
    ct = jnp.dot(cs_ref[...].astype(bf16), t_ref[...],
                 preferred_element_type=f32).astype(bf16)
    msg_ref[...] = jnp.dot(wg_ref[...] * ct, g_ref[...],
                           preferred_element_type=f32)


def _msg(wgall, cs, tmat, gmat):
    eb = 2048
    return pl.pallas_call(
        _msg_body,
        grid=(N_EDGES // eb,),
        in_specs=[
            pl.BlockSpec((eb, D * D), lambda i: (i, 0)),
            pl.BlockSpec((eb, D), lambda i: (i, 0)),
            pl.BlockSpec((D, D * D), lambda i: (0, 0)),
            pl.BlockSpec((D * D, D), lambda i: (0, 0)),
        ],
        out_specs=pl.BlockSpec((eb, D), lambda i: (i, 0)),
        out_shape=jax.ShapeDtypeStruct((N_EDGES, D), jnp.float32),
    )(wgall, cs, tmat, gmat)


def _gru_compute(a0_ref, a1_ref, c0_ref, c1_ref, s_ref, root_ref, cb_ref,
                 wri_ref, wzi_ref, wni_ref, wrh_ref, wzh_ref, wnh_ref,
                 bi_ref, bh_ref):
    f32 = jnp.float32
    cnt = jnp.maximum(c0_ref[...][:, :1] + c1_ref[...][:, :1], 1.0)
    agg = (a0_ref[...] + a1_ref[...]) / cnt
    s = s_ref[...]
    m = agg + jnp.dot(s, root_ref[...], preferred_element_type=f32)
    m = jnp.maximum(m + cb_ref[...], 0.0)
    bi = bi_ref[...]
    bh = bh_ref[...]
    gir = jnp.dot(m, wri_ref[...], preferred_element_type=f32) + bi[:, :D]
    giz = jnp.dot(m, wzi_ref[...], preferred_element_type=f32) + bi[:, D:2 * D]
    gin = jnp.dot(m, wni_ref[...], preferred_element_type=f32) + bi[:, 2 * D:]
    ghr = jnp.dot(s, wrh_ref[...], preferred_element_type=f32) + bh[:, :D]
    ghz = jnp.dot(s, wzh_ref[...], preferred_element_type=f32) + bh[:, D:2 * D]
    ghn = jnp.dot(s, wnh_ref[...], preferred_element_type=f32) + bh[:, 2 * D:]
    r = jax.nn.sigmoid(gir + ghr)
    z = jax.nn.sigmoid(giz + ghz)
    n = jnp.tanh(gin + r * ghn)
    return (1.0 - z) * n + z * s


def _upd_body(*refs):
    out_ref = refs[-1]
    out_ref[...] = _gru_compute(*refs[:-1])


_NODE_SPECS = None


def _gru_specs(nb):
    specs = [pl.BlockSpec((nb, D), lambda i: (i, 0)) for _ in range(5)]
    wspec = [pl.BlockSpec((D, D), lambda i: (0, 0)) for _ in range(7)]
    bspec = [pl.BlockSpec((1, 3 * D), lambda i: (0, 0)) for _ in range(2)]
    cbspec = [pl.BlockSpec((1, D), lambda i: (0, 0))]
    return specs + wspec[:1] + cbspec + wspec[1:] + bspec


def _update(aggp, cntp, s, root, cb, wsplits, bi, bh):
    nb = 4096
    return pl.pallas_call(
        _upd_body,
        grid=(N_NODES // nb,),
        in_specs=_gru_specs(nb),
        out_specs=pl.BlockSpec((nb, D), lambda i: (i, 0)),
        out_shape=jax.ShapeDtypeStruct((N_NODES, D), jnp.float32),
    )(aggp[0], aggp[1], cntp[0], cntp[1], s, root, cb, *wsplits, bi, bh)


def _updhead_body(*refs):
    mu_ref, std_ref = refs[-2], refs[-1]
    gru_refs = refs[:15]
    w1_ref, b1_ref, w2_ref, b2_ref, w3_ref, b3_ref, sel_ref = refs[15:22]
    h = _gru_compute(*gru_refs)
    t1 = jnp.dot(h, w1_ref[...], preferred_element_type=jnp.float32)
    t1 = jnp.maximum(t1 + b1_ref[...], 0.0)
    t2 = jnp.dot(t1, w2_ref[...], preferred_element_type=jnp.float32)
    t2 = jnp.maximum(t2 + b2_ref[...], 0.0)
    mix = (jnp.dot(t2, w3_ref[...], preferred_element_type=jnp.float32)
           + b3_ref[...])
    sel = sel_ref[...]
    lane = lax.broadcasted_iota(jnp.int32, mix.shape, 1)
    oh = (lane == sel).astype(jnp.float32)
    mu_ref[...] = jnp.sum(mix * oh, axis=1, keepdims=True)
    mm = jnp.mean(mix, axis=1, keepdims=True)
    var = jnp.sum((mix - mm) * (mix - mm), axis=1, keepdims=True) / (MIX_N - 1.0)
    std_ref[...] = jnp.sqrt(var + VAR_EPS)


def _updheads(aggp, cntp, s, root, cb, wsplits, bi, bh,
              w1, b1, w2, b2, w3, b3, sel):
    nb = 4096
    kd = MIX_N * D
    head_specs = [
        pl.BlockSpec((D, kd), lambda i: (0, 0)),
        pl.BlockSpec((1, kd), lambda i: (0, 0)),
        pl.BlockSpec((kd, kd), lambda i: (0, 0)),
        pl.BlockSpec((1, kd), lambda i: (0, 0)),
        pl.BlockSpec((kd, MIX_N), lambda i: (0, 0)),
        pl.BlockSpec((1, MIX_N), lambda i: (0, 0)),
        pl.BlockSpec((nb, 1), lambda i: (i, 0)),
    ]
    return pl.pallas_call(
        _updhead_body,
        grid=(N_NODES // nb,),
        in_specs=_gru_specs(nb) + head_specs,
        out_specs=[
            pl.BlockSpec((nb, 1), lambda i: (i, 0)),
            pl.BlockSpec((nb, 1), lambda i: (i, 0)),
        ],
        out_shape=[
            jax.ShapeDtypeStruct((N_NODES, 1), jnp.float32),
            jax.ShapeDtypeStruct((N_NODES, 1), jnp.float32),
        ],
    )(aggp[0], aggp[1], cntp[0], cntp[1], s, root, cb, *wsplits, bi, bh,
      w1, b1, w2, b2, w3, b3, sel)


def _gather_body(cur_hbm, src_hbm, out_hbm, idx_v, rows_v, tab_sh, sem):
    c = lax.axis_index("c")
    s = lax.axis_index("s")
    wid = s * 2 + c
    nps = N_NODES // 16
    pltpu.sync_copy(cur_hbm.at[pl.ds(s * nps, nps)], tab_sh.at[pl.ds(s * nps, nps)])
    pltpu.sync_copy(src_hbm.at[pl.ds(wid * NCH, NCH)], idx_v)
    plsc.subcore_barrier()
    copies = []
    for j in range(NCH):
        copies.append(pltpu.async_copy(
            tab_sh.at[idx_v.at[j]], rows_v.at[pl.ds(j * CHUNK, CHUNK)], sem))
    for cp in copies:
        cp.wait()
    pltpu.sync_copy(rows_v, out_hbm.at[pl.ds(wid * EPW, EPW)])


def _sc_gather(cur, src2d):
    mesh = plsc.VectorSubcoreMesh(core_axis_name="c", subcore_axis_name="s")
    return pl.kernel(
        _gather_body,
        out_type=jax.ShapeDtypeStruct((N_EDGES, D), jnp.float32),
        mesh=mesh,
        compiler_params=pltpu.CompilerParams(use_tc_tiling_on_sc=False),
        scratch_types=[
            pltpu.VMEM((NCH, CHUNK), jnp.int32),
            pltpu.VMEM((EPW, D), jnp.float32),
            pltpu.VMEM_SHARED((N_NODES, D), jnp.float32),
            pltpu.SemaphoreType.DMA,
        ],
    )(cur, src2d)


def _scatter_body(msg_hbm, dst_hbm, zero_hbm, out_hbm, idx_v, rows_v, acc_sh, sem):
    c = lax.axis_index("c")
    s = lax.axis_index("s")
    wid = s * 2 + c
    rps = N_NODES // 16
    pltpu.sync_copy(zero_hbm, acc_sh.at[pl.ds(s * rps, rps)])
    pltpu.sync_copy(dst_hbm.at[pl.ds(wid * NCH, NCH)], idx_v)
    pltpu.sync_copy(msg_hbm.at[pl.ds(wid * EPW, EPW)], rows_v)
    plsc.subcore_barrier()
    for j in range(NCH):
        pltpu.sync_copy(rows_v.at[pl.ds(j * CHUNK, CHUNK)],
                        acc_sh.at[idx_v.at[j]], add=True)
    plsc.subcore_barrier()
    pltpu.sync_copy(acc_sh.at[pl.ds(s * rps, rps)],
                    out_hbm.at[c, pl.ds(s * rps, rps)])


def _sc_scatter(msg, dst2d, zero_rows):
    mesh = plsc.VectorSubcoreMesh(core_axis_name="c", subcore_axis_name="s")
    return pl.kernel(
        _scatter_body,
        out_type=jax.ShapeDtypeStruct((2, N_NODES, D), jnp.float32),
        mesh=mesh,
        compiler_params=pltpu.CompilerParams(use_tc_tiling_on_sc=False),
        scratch_types=[
            pltpu.VMEM((NCH, CHUNK), jnp.int32),
            pltpu.VMEM((EPW, D), jnp.float32),
            pltpu.VMEM_SHARED((N_NODES, D), jnp.float32),
            pltpu.SemaphoreType.DMA,
        ],
    )(msg, dst2d, zero_rows)


def kernel(x, edge_index, edge_attr, input_idx, W_embed, b_embed, bn_gamma,
           bn_beta, We1, be1, We2, be2, root, conv_bias, Wih, Whh, bih, bhh,
           mW1, mb1, mW2, mb2, mW3, mb3):
    f32 = jnp.float32
    src2d = edge_index[0].reshape(N_EDGES // CHUNK, CHUNK)
    dst2d = edge_index[1].reshape(N_EDGES // CHUNK, CHUNK)

    wt = W_embed.T
    b2 = b_embed.reshape(1, D)
    g2 = bn_gamma.reshape(1, D)
    bt2 = bn_beta.reshape(1, D)
    w1t = We1.T
    be1r = be1.reshape(1, EL)
    we2p = We2.reshape(D, D, EL).transpose(1, 0, 2).reshape(D * D, EL)
    w2t = we2p.T.astype(jnp.bfloat16)
    be2p = be2.reshape(D, D).T.reshape(1, D * D)
    eye = jnp.eye(D, dtype=jnp.bfloat16)
    tmat = jnp.tile(eye, (1, D))
    gmat = jnp.repeat(eye, D, axis=0)
    cb = conv_bias.reshape(1, D)
    wsplits = (Wih[:D].T, Wih[D:2 * D].T, Wih[2 * D:].T,
               Whh[:D].T, Whh[D:2 * D].T, Whh[2 * D:].T)
    bi = bih.reshape(1, 3 * D)
    bh = bhh.reshape(1, 3 * D)
    kd = MIX_N * D
    w1 = jnp.transpose(mW1, (2, 0, 1)).reshape(D, kd)
    b1 = mb1.reshape(1, kd)
    w2bd = jax.scipy.linalg.block_diag(*[mW2[k].T for k in range(MIX_N)])
    b2h = mb2.reshape(1, kd)
    w3bd = jax.scipy.linalg.block_diag(*[mW3[k].T for k in range(MIX_N)])
    b3h = mb3.reshape(1, MIX_N)
    sel = jnp.repeat(jnp.mod(input_idx, MIX_N), MAX_N).reshape(N_NODES, 1)
    sel = sel.astype(jnp.int32)
    zero_rows = jnp.zeros((N_NODES // 16, D), f32)
    ones_rows = jnp.ones((N_EDGES, D), f32)

    y, wgall = _pre(x, wt, b2, g2, bt2, edge_attr, w1t, be1r, w2t, be2p)
    cntp = _sc_scatter(ones_rows, dst2d, zero_rows)

    s = y
    for it in range(3):
        cs = _sc_gather(s, src2d)
        msg = _msg(wgall, cs, tmat, gmat)
        aggp = _sc_scatter(msg, dst2d, zero_rows)
        if it < 2:
            s = _update(aggp, cntp, s, root, cb, wsplits, bi, bh)
    mu, std = _updheads(aggp, cntp, s, root, cb, wsplits, bi, bh,
                        w1, b1, w2bd, b2h, w3bd, b3h, sel)
    return (mu.reshape(N_NODES // MAX_N, MAX_N, 1),
            std.reshape(N_NODES // MAX_N, MAX_N, 1))


# speedup vs baseline: 2.6595x; 1.0137x over previous
import functools

import jax
import jax.numpy as jnp
from jax import lax
from jax.experimental import pallas as pl
from jax.experimental.pallas import tpu as pltpu
from jax.experimental.pallas import tpu_sc as plsc

N_NODES = 16384
N_EDGES = 65536
D_IN = 128
D = 32
EL = 128
MIX_N = 10
MAX_N = 64
BN_EPS = 1e-5
VAR_EPS = 1e-5

NW = 32
EPW = N_EDGES // NW
CHUNK = 128
NCH = EPW // CHUNK


def _embed_body(x_ref, wt_ref, b_ref, g_ref, bt_ref, y_ref):
    y = jnp.dot(x_ref[...], wt_ref[...], preferred_element_type=jnp.float32)
    y = y + b_ref[...]
    mean = jnp.mean(y, axis=0, keepdims=True)
    var = jnp.mean((y - mean) * (y - mean), axis=0, keepdims=True)
    y_ref[...] = ((y - mean) * lax.rsqrt(var + BN_EPS) * g_ref[...]
                  + bt_ref[...])


def _embed(x, wt, b, g, bt):
    return pl.pallas_call(
        _embed_body,
        out_shape=jax.ShapeDtypeStruct((N_NODES, D), jnp.float32),
    )(x, wt, b, g, bt)


def _wg_body(ea_ref, w1t_ref, b1_ref, w2_ref, b2_ref, wg_ref):
    t = jnp.dot(ea_ref[...], w1t_ref[...], preferred_element_type=jnp.float32)
    eh = jax.nn.sigmoid(t + b1_ref[...]).astype(jnp.bfloat16)
    wg = jnp.dot(eh, w2_ref[...], preferred_element_type=jnp.float32)
    wg_ref[...] = (wg + b2_ref[...]).astype(jnp.bfloat16)


def _wgpre(ea, w1t, b1, w2t, b2p):
    eb = 2048
    return pl.pallas_call(
        _wg_body,
        grid=(N_EDGES // eb,),
        in_specs=[
            pl.BlockSpec((eb, 4), lambda i: (i, 0)),
            pl.BlockSpec((4, EL), lambda i: (0, 0)),
            pl.BlockSpec((1, EL), lambda i: (0, 0)),
            pl.BlockSpec((EL, D * D), lambda i: (0, 0)),
            pl.BlockSpec((1, D * D), lambda i: (0, 0)),
        ],
        out_specs=pl.BlockSpec((eb, D * D), lambda i: (i, 0)),
        out_shape=jax.ShapeDtypeStruct((N_EDGES, D * D), jnp.bfloat16),
    )(ea, w1t, b1, w2t, b2p)


def _msg_body(wg_ref, cs_ref, t_ref, g_ref, msg_ref):
    f32 = jnp.float32
    bf16 = jnp.bfloat16
    ct = jnp.dot(cs_ref[...].astype(bf16), t_ref[...],
                 preferred_element_type=f32).astype(bf16)
    msg_ref[...] = jnp.dot(wg_ref[...] * ct, g_ref[...],
                           preferred_element_type=f32)


def _msg(wgall, cs, tmat, gmat):
    eb = 2048
    return pl.pallas_call(
        _msg_body,
        grid=(N_EDGES // eb,),
        in_specs=[
            pl.BlockSpec((eb, D * D), lambda i: (i, 0)),
            pl.BlockSpec((eb, D), lambda i: (i, 0)),
            pl.BlockSpec((D, D * D), lambda i: (0, 0)),
            pl.BlockSpec((D * D, D), lambda i: (0, 0)),
        ],
        out_specs=pl.BlockSpec((eb, D), lambda i: (i, 0)),
        out_shape=jax.ShapeDtypeStruct((N_EDGES, D), jnp.float32),
    )(wgall, cs, tmat, gmat)


def _gru_compute(a0_ref, a1_ref, c0_ref, c1_ref, s_ref, root_ref, cb_ref,
                 wri_ref, wzi_ref, wni_ref, wrh_ref, wzh_ref, wnh_ref,
                 bi_ref, bh_ref):
    f32 = jnp.float32
    cnt = jnp.maximum(c0_ref[...][:, :1] + c1_ref[...][:, :1], 1.0)
    agg = (a0_ref[...] + a1_ref[...]) / cnt
    s = s_ref[...]
    m = agg + jnp.dot(s, root_ref[...], preferred_element_type=f32)
    m = jnp.maximum(m + cb_ref[...], 0.0)
    bi = bi_ref[...]
    bh = bh_ref[...]
    gir = jnp.dot(m, wri_ref[...], preferred_element_type=f32) + bi[:, :D]
    giz = jnp.dot(m, wzi_ref[...], preferred_element_type=f32) + bi[:, D:2 * D]
    gin = jnp.dot(m, wni_ref[...], preferred_element_type=f32) + bi[:, 2 * D:]
    ghr = jnp.dot(s, wrh_ref[...], preferred_element_type=f32) + bh[:, :D]
    ghz = jnp.dot(s, wzh_ref[...], preferred_element_type=f32) + bh[:, D:2 * D]
    ghn = jnp.dot(s, wnh_ref[...], preferred_element_type=f32) + bh[:, 2 * D:]
    r = jax.nn.sigmoid(gir + ghr)
    z = jax.nn.sigmoid(giz + ghz)
    n = jnp.tanh(gin + r * ghn)
    return (1.0 - z) * n + z * s


def _upd_body(*refs):
    out_ref = refs[-1]
    out_ref[...] = _gru_compute(*refs[:-1])


_NODE_SPECS = None


def _gru_specs(nb):
    specs = [pl.BlockSpec((nb, D), lambda i: (i, 0)) for _ in range(5)]
    wspec = [pl.BlockSpec((D, D), lambda i: (0, 0)) for _ in range(7)]
    bspec = [pl.BlockSpec((1, 3 * D), lambda i: (0, 0)) for _ in range(2)]
    cbspec = [pl.BlockSpec((1, D), lambda i: (0, 0))]
    return specs + wspec[:1] + cbspec + wspec[1:] + bspec


def _update(aggp, cntp, s, root, cb, wsplits, bi, bh):
    nb = 4096
    return pl.pallas_call(
        _upd_body,
        grid=(N_NODES // nb,),
        in_specs=_gru_specs(nb),
        out_specs=pl.BlockSpec((nb, D), lambda i: (i, 0)),
        out_shape=jax.ShapeDtypeStruct((N_NODES, D), jnp.float32),
    )(aggp[0], aggp[1], cntp[0], cntp[1], s, root, cb, *wsplits, bi, bh)


def _updhead_body(*refs):
    mu_ref, std_ref = refs[-2], refs[-1]
    gru_refs = refs[:15]
    w1_ref, b1_ref, w2_ref, b2_ref, w3_ref, b3_ref, sel_ref = refs[15:22]
    h = _gru_compute(*gru_refs)
    t1 = jnp.dot(h, w1_ref[...], preferred_element_type=jnp.float32)
    t1 = jnp.maximum(t1 + b1_ref[...], 0.0)
    t2 = jnp.dot(t1, w2_ref[...], preferred_element_type=jnp.float32)
    t2 = jnp.maximum(t2 + b2_ref[...], 0.0)
    mix = (jnp.dot(t2, w3_ref[...], preferred_element_type=jnp.float32)
           + b3_ref[...])
    sel = sel_ref[...]
    lane = lax.broadcasted_iota(jnp.int32, mix.shape, 1)
    oh = (lane == sel).astype(jnp.float32)
    mu_ref[...] = jnp.sum(mix * oh, axis=1, keepdims=True)
    mm = jnp.mean(mix, axis=1, keepdims=True)
    var = jnp.sum((mix - mm) * (mix - mm), axis=1, keepdims=True) / (MIX_N - 1.0)
    std_ref[...] = jnp.sqrt(var + VAR_EPS)


def _updheads(aggp, cntp, s, root, cb, wsplits, bi, bh,
              w1, b1, w2, b2, w3, b3, sel):
    nb = 4096
    kd = MIX_N * D
    head_specs = [
        pl.BlockSpec((D, kd), lambda i: (0, 0)),
        pl.BlockSpec((1, kd), lambda i: (0, 0)),
        pl.BlockSpec((kd, kd), lambda i: (0, 0)),
        pl.BlockSpec((1, kd), lambda i: (0, 0)),
        pl.BlockSpec((kd, MIX_N), lambda i: (0, 0)),
        pl.BlockSpec((1, MIX_N), lambda i: (0, 0)),
        pl.BlockSpec((nb, 1), lambda i: (i, 0)),
    ]
    return pl.pallas_call(
        _updhead_body,
        grid=(N_NODES // nb,),
        in_specs=_gru_specs(nb) + head_specs,
        out_specs=[
            pl.BlockSpec((nb, 1), lambda i: (i, 0)),
            pl.BlockSpec((nb, 1), lambda i: (i, 0)),
        ],
        out_shape=[
            jax.ShapeDtypeStruct((N_NODES, 1), jnp.float32),
            jax.ShapeDtypeStruct((N_NODES, 1), jnp.float32),
        ],
    )(aggp[0], aggp[1], cntp[0], cntp[1], s, root, cb, *wsplits, bi, bh,
      w1, b1, w2, b2, w3, b3, sel)


def _gather_body(cur_hbm, src_hbm, out_hbm, idx_v, rows_v, tab_sh, sem):
    c = lax.axis_index("c")
    s = lax.axis_index("s")
    wid = s * 2 + c
    nps = N_NODES // 16
    pltpu.sync_copy(cur_hbm.at[pl.ds(s * nps, nps)], tab_sh.at[pl.ds(s * nps, nps)])
    pltpu.sync_copy(src_hbm.at[pl.ds(wid * NCH, NCH)], idx_v)
    plsc.subcore_barrier()
    copies = []
    for j in range(NCH):
        copies.append(pltpu.async_copy(
            tab_sh.at[idx_v.at[j]], rows_v.at[pl.ds(j * CHUNK, CHUNK)], sem))
    for cp in copies:
        cp.wait()
    pltpu.sync_copy(rows_v, out_hbm.at[pl.ds(wid * EPW, EPW)])


def _sc_gather(cur, src2d):
    mesh = plsc.VectorSubcoreMesh(core_axis_name="c", subcore_axis_name="s")
    return pl.kernel(
        _gather_body,
        out_type=jax.ShapeDtypeStruct((N_EDGES, D), jnp.float32),
        mesh=mesh,
        compiler_params=pltpu.CompilerParams(use_tc_tiling_on_sc=False),
        scratch_types=[
            pltpu.VMEM((NCH, CHUNK), jnp.int32),
            pltpu.VMEM((EPW, D), jnp.float32),
            pltpu.VMEM_SHARED((N_NODES, D), jnp.float32),
            pltpu.SemaphoreType.DMA,
        ],
    )(cur, src2d)


def _scatter_body(msg_hbm, dst_hbm, zero_hbm, out_hbm, idx_v, rows_v, acc_sh, sem):
    c = lax.axis_index("c")
    s = lax.axis_index("s")
    wid = s * 2 + c
    rps = N_NODES // 16
    pltpu.sync_copy(zero_hbm, acc_sh.at[pl.ds(s * rps, rps)])
    pltpu.sync_copy(dst_hbm.at[pl.ds(wid * NCH, NCH)], idx_v)
    pltpu.sync_copy(msg_hbm.at[pl.ds(wid * EPW, EPW)], rows_v)
    plsc.subcore_barrier()
    for j in range(NCH):
        pltpu.sync_copy(rows_v.at[pl.ds(j * CHUNK, CHUNK)],
                        acc_sh.at[idx_v.at[j]], add=True)
    plsc.subcore_barrier()
    pltpu.sync_copy(acc_sh.at[pl.ds(s * rps, rps)],
                    out_hbm.at[c, pl.ds(s * rps, rps)])


def _sc_scatter(msg, dst2d, zero_rows):
    mesh = plsc.VectorSubcoreMesh(core_axis_name="c", subcore_axis_name="s")
    return pl.kernel(
        _scatter_body,
        out_type=jax.ShapeDtypeStruct((2, N_NODES, D), jnp.float32),
        mesh=mesh,
        compiler_params=pltpu.CompilerParams(use_tc_tiling_on_sc=False),
        scratch_types=[
            pltpu.VMEM((NCH, CHUNK), jnp.int32),
            pltpu.VMEM((EPW, D), jnp.float32),
            pltpu.VMEM_SHARED((N_NODES, D), jnp.float32),
            pltpu.SemaphoreType.DMA,
        ],
    )(msg, dst2d, zero_rows)


def kernel(x, edge_index, edge_attr, input_idx, W_embed, b_embed, bn_gamma,
           bn_beta, We1, be1, We2, be2, root, conv_bias, Wih, Whh, bih, bhh,
           mW1, mb1, mW2, mb2, mW3, mb3):
    f32 = jnp.float32
    src2d = edge_index[0].reshape(N_EDGES // CHUNK, CHUNK)
    dst2d = edge_index[1].reshape(N_EDGES // CHUNK, CHUNK)

    wt = W_embed.T
    b2 = b_embed.reshape(1, D)
    g2 = bn_gamma.reshape(1, D)
    bt2 = bn_beta.reshape(1, D)
    w1t = We1.T
    be1r = be1.reshape(1, EL)
    we2p = We2.reshape(D, D, EL).transpose(1, 0, 2).reshape(D * D, EL)
    w2t = we2p.T.astype(jnp.bfloat16)
    be2p = be2.reshape(D, D).T.reshape(1, D * D)
    eye = jnp.eye(D, dtype=jnp.bfloat16)
    tmat = jnp.tile(eye, (1, D))
    gmat = jnp.repeat(eye, D, axis=0)
    cb = conv_bias.reshape(1, D)
    wsplits = (Wih[:D].T, Wih[D:2 * D].T, Wih[2 * D:].T,
               Whh[:D].T, Whh[D:2 * D].T, Whh[2 * D:].T)
    bi = bih.reshape(1, 3 * D)
    bh = bhh.reshape(1, 3 * D)
    kd = MIX_N * D
    w1 = jnp.transpose(mW1, (2, 0, 1)).reshape(D, kd)
    b1 = mb1.reshape(1, kd)
    w2bd = jax.scipy.linalg.block_diag(*[mW2[k].T for k in range(MIX_N)])
    b2h = mb2.reshape(1, kd)
    w3bd = jax.scipy.linalg.block_diag(*[mW3[k].T for k in range(MIX_N)])
    b3h = mb3.reshape(1, MIX_N)
    sel = jnp.repeat(jnp.mod(input_idx, MIX_N), MAX_N).reshape(N_NODES, 1)
    sel = sel.astype(jnp.int32)
    zero_rows = jnp.zeros((N_NODES // 16, D), f32)
    ones_rows = jnp.ones((N_EDGES, D), f32)

    y = _embed(x, wt, b2, g2, bt2)
    wgall = _wgpre(edge_attr, w1t, be1r, w2t, be2p)
    cntp = _sc_scatter(ones_rows, dst2d, zero_rows)

    s = y
    for it in range(3):
        cs = _sc_gather(s, src2d)
        msg = _msg(wgall, cs, tmat, gmat)
        aggp = _sc_scatter(msg, dst2d, zero_rows)
        if it < 2:
            s = _update(aggp, cntp, s, root, cb, wsplits, bi, bh)
    mu, std = _updheads(aggp, cntp, s, root, cb, wsplits, bi, bh,
                        w1, b1, w2bd, b2h, w3bd, b3h, sel)
    return (mu.reshape(N_NODES // MAX_N, MAX_N, 1),
            std.reshape(N_NODES // MAX_N, MAX_N, 1))


# speedup vs baseline: 3.1575x; 1.1873x over previous
import functools

import jax
import jax.numpy as jnp
from jax import lax
from jax.experimental import pallas as pl
from jax.experimental.pallas import tpu as pltpu
from jax.experimental.pallas import tpu_sc as plsc

N_NODES = 16384
N_EDGES = 65536
D_IN = 128
D = 32
EL = 128
MIX_N = 10
MAX_N = 64
BN_EPS = 1e-5
VAR_EPS = 1e-5

NW = 32
EPW = N_EDGES // NW
CHUNK = 128
NCH = EPW // CHUNK


def _embed_body(x_ref, wt_ref, b_ref, g_ref, bt_ref, y_ref):
    y = jnp.dot(x_ref[...], wt_ref[...], preferred_element_type=jnp.float32)
    y = y + b_ref[...]
    mean = jnp.mean(y, axis=0, keepdims=True)
    var = jnp.mean((y - mean) * (y - mean), axis=0, keepdims=True)
    y_ref[...] = ((y - mean) * lax.rsqrt(var + BN_EPS) * g_ref[...]
                  + bt_ref[...])


def _embed(x, wt, b, g, bt):
    return pl.pallas_call(
        _embed_body,
        out_shape=jax.ShapeDtypeStruct((N_NODES, D), jnp.float32),
    )(x, wt, b, g, bt)


def _wg_body(ea4_ref, w1t_ref, b1_ref, w2_ref, b2_ref, wg4_ref):
    f32 = jnp.float32
    ea4 = ea4_ref[...]
    parts = []
    for q in range(4):
        t = jnp.dot(ea4[:, q * 4:(q + 1) * 4], w1t_ref[...],
                    preferred_element_type=f32)
        eh = jax.nn.sigmoid(t + b1_ref[...]).astype(jnp.bfloat16)
        wg = jnp.dot(eh, w2_ref[...], preferred_element_type=f32)
        parts.append((wg + b2_ref[...]).astype(jnp.bfloat16))
    wg4_ref[...] = jnp.concatenate(parts, axis=1)


def _wgpre(ea4, w1t, b1, w2t, b2p):
    eb4 = 512
    return pl.pallas_call(
        _wg_body,
        grid=(N_EDGES // 4 // eb4,),
        in_specs=[
            pl.BlockSpec((eb4, 16), lambda i: (i, 0)),
            pl.BlockSpec((4, EL), lambda i: (0, 0)),
            pl.BlockSpec((1, EL), lambda i: (0, 0)),
            pl.BlockSpec((EL, D * D), lambda i: (0, 0)),
            pl.BlockSpec((1, D * D), lambda i: (0, 0)),
        ],
        out_specs=pl.BlockSpec((eb4, 4 * D * D), lambda i: (i, 0)),
        out_shape=jax.ShapeDtypeStruct((N_EDGES // 4, 4 * D * D),
                                       jnp.bfloat16),
    )(ea4, w1t, b1, w2t, b2p)


def _msg_body(wg4_ref, cs4_ref, t_ref, g_ref, msg4_ref):
    f32 = jnp.float32
    bf16 = jnp.bfloat16
    cs4 = cs4_ref[...].astype(bf16)
    wg4 = wg4_ref[...]
    dd = D * D
    parts = []
    for q in range(4):
        ct = jnp.dot(cs4[:, q * D:(q + 1) * D], t_ref[...],
                     preferred_element_type=f32).astype(bf16)
        parts.append(jnp.dot(wg4[:, q * dd:(q + 1) * dd] * ct, g_ref[...],
                             preferred_element_type=f32))
    msg4_ref[...] = jnp.concatenate(parts, axis=1)


def _msg(wg4, cs4, tmat, gmat):
    eb4 = 512
    return pl.pallas_call(
        _msg_body,
        grid=(N_EDGES // 4 // eb4,),
        in_specs=[
            pl.BlockSpec((eb4, 4 * D * D), lambda i: (i, 0)),
            pl.BlockSpec((eb4, 4 * D), lambda i: (i, 0)),
            pl.BlockSpec((D, D * D), lambda i: (0, 0)),
            pl.BlockSpec((D * D, D), lambda i: (0, 0)),
        ],
        out_specs=pl.BlockSpec((eb4, 4 * D), lambda i: (i, 0)),
        out_shape=jax.ShapeDtypeStruct((N_EDGES // 4, 4 * D), jnp.float32),
    )(wg4, cs4, tmat, gmat)


def _gru_compute(a0_ref, a1_ref, c0_ref, c1_ref, s_ref, root_ref, cb_ref,
                 wri_ref, wzi_ref, wni_ref, wrh_ref, wzh_ref, wnh_ref,
                 bi_ref, bh_ref):
    f32 = jnp.float32
    cnt = jnp.maximum(c0_ref[...][:, :1] + c1_ref[...][:, :1], 1.0)
    agg = (a0_ref[...] + a1_ref[...]) / cnt
    s = s_ref[...]
    m = agg + jnp.dot(s, root_ref[...], preferred_element_type=f32)
    m = jnp.maximum(m + cb_ref[...], 0.0)
    bi = bi_ref[...]
    bh = bh_ref[...]
    gir = jnp.dot(m, wri_ref[...], preferred_element_type=f32) + bi[:, :D]
    giz = jnp.dot(m, wzi_ref[...], preferred_element_type=f32) + bi[:, D:2 * D]
    gin = jnp.dot(m, wni_ref[...], preferred_element_type=f32) + bi[:, 2 * D:]
    ghr = jnp.dot(s, wrh_ref[...], preferred_element_type=f32) + bh[:, :D]
    ghz = jnp.dot(s, wzh_ref[...], preferred_element_type=f32) + bh[:, D:2 * D]
    ghn = jnp.dot(s, wnh_ref[...], preferred_element_type=f32) + bh[:, 2 * D:]
    r = jax.nn.sigmoid(gir + ghr)
    z = jax.nn.sigmoid(giz + ghz)
    n = jnp.tanh(gin + r * ghn)
    return (1.0 - z) * n + z * s


def _upd_body(*refs):
    out_ref = refs[-1]
    out_ref[...] = _gru_compute(*refs[:-1])


_NODE_SPECS = None


def _gru_specs(nb):
    specs = [pl.BlockSpec((nb, D), lambda i: (i, 0)) for _ in range(5)]
    wspec = [pl.BlockSpec((D, D), lambda i: (0, 0)) for _ in range(7)]
    bspec = [pl.BlockSpec((1, 3 * D), lambda i: (0, 0)) for _ in range(2)]
    cbspec = [pl.BlockSpec((1, D), lambda i: (0, 0))]
    return specs + wspec[:1] + cbspec + wspec[1:] + bspec


def _update(aggp, cntp, s, root, cb, wsplits, bi, bh):
    nb = 4096
    return pl.pallas_call(
        _upd_body,
        grid=(N_NODES // nb,),
        in_specs=_gru_specs(nb),
        out_specs=pl.BlockSpec((nb, D), lambda i: (i, 0)),
        out_shape=jax.ShapeDtypeStruct((N_NODES, D), jnp.float32),
    )(aggp[0], aggp[1], cntp[0], cntp[1], s, root, cb, *wsplits, bi, bh)


def _updhead_body(*refs):
    mu_ref, std_ref = refs[-2], refs[-1]
    gru_refs = refs[:15]
    w1_ref, b1_ref, w2_ref, b2_ref, w3_ref, b3_ref, sel_ref = refs[15:22]
    h = _gru_compute(*gru_refs)
    t1 = jnp.dot(h, w1_ref[...], preferred_element_type=jnp.float32)
    t1 = jnp.maximum(t1 + b1_ref[...], 0.0)
    t2 = jnp.dot(t1, w2_ref[...], preferred_element_type=jnp.float32)
    t2 = jnp.maximum(t2 + b2_ref[...], 0.0)
    mix = (jnp.dot(t2, w3_ref[...], preferred_element_type=jnp.float32)
           + b3_ref[...])
    sel = sel_ref[...]
    lane = lax.broadcasted_iota(jnp.int32, mix.shape, 1)
    oh = (lane == sel).astype(jnp.float32)
    mu_ref[...] = jnp.sum(mix * oh, axis=1, keepdims=True)
    mm = jnp.mean(mix, axis=1, keepdims=True)
    var = jnp.sum((mix - mm) * (mix - mm), axis=1, keepdims=True) / (MIX_N - 1.0)
    std_ref[...] = jnp.sqrt(var + VAR_EPS)


def _updheads(aggp, cntp, s, root, cb, wsplits, bi, bh,
              w1, b1, w2, b2, w3, b3, sel):
    nb = 4096
    kd = MIX_N * D
    head_specs = [
        pl.BlockSpec((D, kd), lambda i: (0, 0)),
        pl.BlockSpec((1, kd), lambda i: (0, 0)),
        pl.BlockSpec((kd, kd), lambda i: (0, 0)),
        pl.BlockSpec((1, kd), lambda i: (0, 0)),
        pl.BlockSpec((kd, MIX_N), lambda i: (0, 0)),
        pl.BlockSpec((1, MIX_N), lambda i: (0, 0)),
        pl.BlockSpec((nb, 1), lambda i: (i, 0)),
    ]
    return pl.pallas_call(
        _updhead_body,
        grid=(N_NODES // nb,),
        in_specs=_gru_specs(nb) + head_specs,
        out_specs=[
            pl.BlockSpec((nb, 1), lambda i: (i, 0)),
            pl.BlockSpec((nb, 1), lambda i: (i, 0)),
        ],
        out_shape=[
            jax.ShapeDtypeStruct((N_NODES, 1), jnp.float32),
            jax.ShapeDtypeStruct((N_NODES, 1), jnp.float32),
        ],
    )(aggp[0], aggp[1], cntp[0], cntp[1], s, root, cb, *wsplits, bi, bh,
      w1, b1, w2, b2, w3, b3, sel)


def _gather_body(cur_hbm, src_hbm, out_hbm, idx_v, rows_v, tab_sh, sem):
    c = lax.axis_index("c")
    s = lax.axis_index("s")
    wid = s * 2 + c
    nps = N_NODES // 16
    pltpu.sync_copy(cur_hbm.at[pl.ds(s * nps, nps)], tab_sh.at[pl.ds(s * nps, nps)])
    pltpu.sync_copy(src_hbm.at[pl.ds(wid * NCH, NCH)], idx_v)
    plsc.subcore_barrier()
    copies = []
    for j in range(NCH):
        copies.append(pltpu.async_copy(
            tab_sh.at[idx_v.at[j]], rows_v.at[pl.ds(j * CHUNK, CHUNK)], sem))
    for cp in copies:
        cp.wait()
    pltpu.sync_copy(rows_v, out_hbm.at[pl.ds(wid * EPW, EPW)])


def _sc_gather(cur, src2d):
    mesh = plsc.VectorSubcoreMesh(core_axis_name="c", subcore_axis_name="s")
    return pl.kernel(
        _gather_body,
        out_type=jax.ShapeDtypeStruct((N_EDGES, D), jnp.float32),
        mesh=mesh,
        compiler_params=pltpu.CompilerParams(use_tc_tiling_on_sc=False),
        scratch_types=[
            pltpu.VMEM((NCH, CHUNK), jnp.int32),
            pltpu.VMEM((EPW, D), jnp.float32),
            pltpu.VMEM_SHARED((N_NODES, D), jnp.float32),
            pltpu.SemaphoreType.DMA,
        ],
    )(cur, src2d)


def _scatter_body(msg_hbm, dst_hbm, zero_hbm, out_hbm, idx_v, rows_v, acc_sh, sem):
    c = lax.axis_index("c")
    s = lax.axis_index("s")
    wid = s * 2 + c
    rps = N_NODES // 16
    pltpu.sync_copy(zero_hbm, acc_sh.at[pl.ds(s * rps, rps)])
    pltpu.sync_copy(dst_hbm.at[pl.ds(wid * NCH, NCH)], idx_v)
    pltpu.sync_copy(msg_hbm.at[pl.ds(wid * EPW, EPW)], rows_v)
    plsc.subcore_barrier()
    for j in range(NCH):
        pltpu.sync_copy(rows_v.at[pl.ds(j * CHUNK, CHUNK)],
                        acc_sh.at[idx_v.at[j]], add=True)
    plsc.subcore_barrier()
    pltpu.sync_copy(acc_sh.at[pl.ds(s * rps, rps)],
                    out_hbm.at[c, pl.ds(s * rps, rps)])


def _sc_scatter(msg, dst2d, zero_rows):
    mesh = plsc.VectorSubcoreMesh(core_axis_name="c", subcore_axis_name="s")
    return pl.kernel(
        _scatter_body,
        out_type=jax.ShapeDtypeStruct((2, N_NODES, D), jnp.float32),
        mesh=mesh,
        compiler_params=pltpu.CompilerParams(use_tc_tiling_on_sc=False),
        scratch_types=[
            pltpu.VMEM((NCH, CHUNK), jnp.int32),
            pltpu.VMEM((EPW, D), jnp.float32),
            pltpu.VMEM_SHARED((N_NODES, D), jnp.float32),
            pltpu.SemaphoreType.DMA,
        ],
    )(msg, dst2d, zero_rows)


def kernel(x, edge_index, edge_attr, input_idx, W_embed, b_embed, bn_gamma,
           bn_beta, We1, be1, We2, be2, root, conv_bias, Wih, Whh, bih, bhh,
           mW1, mb1, mW2, mb2, mW3, mb3):
    f32 = jnp.float32
    src2d = edge_index[0].reshape(N_EDGES // CHUNK, CHUNK)
    dst2d = edge_index[1].reshape(N_EDGES // CHUNK, CHUNK)

    wt = W_embed.T
    b2 = b_embed.reshape(1, D)
    g2 = bn_gamma.reshape(1, D)
    bt2 = bn_beta.reshape(1, D)
    w1t = We1.T
    be1r = be1.reshape(1, EL)
    we2p = We2.reshape(D, D, EL).transpose(1, 0, 2).reshape(D * D, EL)
    w2t = we2p.T.astype(jnp.bfloat16)
    be2p = be2.reshape(D, D).T.reshape(1, D * D)
    eye = jnp.eye(D, dtype=jnp.bfloat16)
    tmat = jnp.tile(eye, (1, D))
    gmat = jnp.repeat(eye, D, axis=0)
    cb = conv_bias.reshape(1, D)
    wsplits = (Wih[:D].T, Wih[D:2 * D].T, Wih[2 * D:].T,
               Whh[:D].T, Whh[D:2 * D].T, Whh[2 * D:].T)
    bi = bih.reshape(1, 3 * D)
    bh = bhh.reshape(1, 3 * D)
    kd = MIX_N * D
    w1 = jnp.transpose(mW1, (2, 0, 1)).reshape(D, kd)
    b1 = mb1.reshape(1, kd)
    w2bd = jax.scipy.linalg.block_diag(*[mW2[k].T for k in range(MIX_N)])
    b2h = mb2.reshape(1, kd)
    w3bd = jax.scipy.linalg.block_diag(*[mW3[k].T for k in range(MIX_N)])
    b3h = mb3.reshape(1, MIX_N)
    sel = jnp.repeat(jnp.mod(input_idx, MIX_N), MAX_N).reshape(N_NODES, 1)
    sel = sel.astype(jnp.int32)
    zero_rows = jnp.zeros((N_NODES // 16, D), f32)
    ones_rows = jnp.ones((N_EDGES, D), f32)

    y = _embed(x, wt, b2, g2, bt2)
    wg4 = _wgpre(edge_attr.reshape(N_EDGES // 4, 16), w1t, be1r, w2t, be2p)
    cntp = _sc_scatter(ones_rows, dst2d, zero_rows)

    s = y
    for it in range(3):
        cs = _sc_gather(s, src2d)
        msg4 = _msg(wg4, cs.reshape(N_EDGES // 4, 4 * D), tmat, gmat)
        aggp = _sc_scatter(msg4.reshape(N_EDGES, D), dst2d, zero_rows)
        if it < 2:
            s = _update(aggp, cntp, s, root, cb, wsplits, bi, bh)
    mu, std = _updheads(aggp, cntp, s, root, cb, wsplits, bi, bh,
                        w1, b1, w2bd, b2h, w3bd, b3h, sel)
    return (mu.reshape(N_NODES // MAX_N, MAX_N, 1),
            std.reshape(N_NODES // MAX_N, MAX_N, 1))


# speedup vs baseline: 3.7786x; 1.1967x over previous
import functools

import jax
import jax.numpy as jnp
from jax import lax
from jax.experimental import pallas as pl
from jax.experimental.pallas import tpu as pltpu
from jax.experimental.pallas import tpu_sc as plsc

N_NODES = 16384
N_EDGES = 65536
D_IN = 128
D = 32
EL = 128
MIX_N = 10
MAX_N = 64
BN_EPS = 1e-5
VAR_EPS = 1e-5

NW = 32
EPW = N_EDGES // NW
CHUNK = 128
NCH = EPW // CHUNK


def _embed_body(x4_ref, wt_ref, b_ref, g_ref, bt_ref, y4_ref):
    f32 = jnp.float32
    x4 = x4_ref[...]
    ys = []
    for q in range(4):
        y = jnp.dot(x4[:, q * D_IN:(q + 1) * D_IN], wt_ref[...],
                    preferred_element_type=f32)
        ys.append(y + b_ref[...])
    sums = sum(jnp.sum(y, axis=0, keepdims=True) for y in ys)
    sqs = sum(jnp.sum(y * y, axis=0, keepdims=True) for y in ys)
    mean = sums / float(N_NODES)
    var = sqs / float(N_NODES) - mean * mean
    scale = lax.rsqrt(var + BN_EPS) * g_ref[...]
    y4_ref[...] = jnp.concatenate(
        [(y - mean) * scale + bt_ref[...] for y in ys], axis=1)


def _embed(x4, wt, b, g, bt):
    return pl.pallas_call(
        _embed_body,
        out_shape=jax.ShapeDtypeStruct((N_NODES // 4, 4 * D), jnp.float32),
    )(x4, wt, b, g, bt)


def _wg_body(ea4_ref, w1t_ref, b1_ref, w2_ref, b2_ref, wg4_ref):
    f32 = jnp.float32
    ea4 = ea4_ref[...]
    parts = []
    for q in range(4):
        t = jnp.dot(ea4[:, q * 4:(q + 1) * 4], w1t_ref[...],
                    preferred_element_type=f32)
        eh = jax.nn.sigmoid(t + b1_ref[...]).astype(jnp.bfloat16)
        wg = jnp.dot(eh, w2_ref[...], preferred_element_type=f32)
        parts.append((wg + b2_ref[...]).astype(jnp.bfloat16))
    wg4_ref[...] = jnp.concatenate(parts, axis=1)


def _wgpre(ea4, w1t, b1, w2t, b2p):
    eb4 = 512
    return pl.pallas_call(
        _wg_body,
        grid=(N_EDGES // 4 // eb4,),
        in_specs=[
            pl.BlockSpec((eb4, 16), lambda i: (i, 0)),
            pl.BlockSpec((4, EL), lambda i: (0, 0)),
            pl.BlockSpec((1, EL), lambda i: (0, 0)),
            pl.BlockSpec((EL, D * D), lambda i: (0, 0)),
            pl.BlockSpec((1, D * D), lambda i: (0, 0)),
        ],
        out_specs=pl.BlockSpec((eb4, 4 * D * D), lambda i: (i, 0)),
        out_shape=jax.ShapeDtypeStruct((N_EDGES // 4, 4 * D * D),
                                       jnp.bfloat16),
    )(ea4, w1t, b1, w2t, b2p)


def _msg_body(wg4_ref, cs4_ref, t_ref, g_ref, msg4_ref):
    f32 = jnp.float32
    bf16 = jnp.bfloat16
    cs4 = cs4_ref[...].astype(bf16)
    wg4 = wg4_ref[...]
    dd = D * D
    parts = []
    for q in range(4):
        ct = jnp.dot(cs4[:, q * D:(q + 1) * D], t_ref[...],
                     preferred_element_type=f32).astype(bf16)
        parts.append(jnp.dot(wg4[:, q * dd:(q + 1) * dd] * ct, g_ref[...],
                             preferred_element_type=f32))
    msg4_ref[...] = jnp.concatenate(parts, axis=1)


def _msg(wg4, cs4, tmat, gmat):
    eb4 = 512
    return pl.pallas_call(
        _msg_body,
        grid=(N_EDGES // 4 // eb4,),
        in_specs=[
            pl.BlockSpec((eb4, 4 * D * D), lambda i: (i, 0)),
            pl.BlockSpec((eb4, 4 * D), lambda i: (i, 0)),
            pl.BlockSpec((D, D * D), lambda i: (0, 0)),
            pl.BlockSpec((D * D, D), lambda i: (0, 0)),
        ],
        out_specs=pl.BlockSpec((eb4, 4 * D), lambda i: (i, 0)),
        out_shape=jax.ShapeDtypeStruct((N_EDGES // 4, 4 * D), jnp.float32),
    )(wg4, cs4, tmat, gmat)


def _gru_compute(ap_ref, cp_ref, s_ref, root_ref, cb_ref,
                 wri_ref, wzi_ref, wni_ref, wrh_ref, wzh_ref, wnh_ref,
                 bi_ref, bh_ref):
    f32 = jnp.float32
    cnt = jnp.maximum(cp_ref[0] + cp_ref[1], 1.0)
    agg4 = (ap_ref[0] + ap_ref[1]) / cnt
    s4 = s_ref[...]
    bi = bi_ref[...]
    bh = bh_ref[...]
    outs = []
    for q in range(4):
        s = s4[:, q * D:(q + 1) * D]
        agg = agg4[:, q * D:(q + 1) * D]
        m = agg + jnp.dot(s, root_ref[...], preferred_element_type=f32)
        m = jnp.maximum(m + cb_ref[...], 0.0)
        gir = jnp.dot(m, wri_ref[...], preferred_element_type=f32) + bi[:, :D]
        giz = jnp.dot(m, wzi_ref[...], preferred_element_type=f32) + bi[:, D:2 * D]
        gin = jnp.dot(m, wni_ref[...], preferred_element_type=f32) + bi[:, 2 * D:]
        ghr = jnp.dot(s, wrh_ref[...], preferred_element_type=f32) + bh[:, :D]
        ghz = jnp.dot(s, wzh_ref[...], preferred_element_type=f32) + bh[:, D:2 * D]
        ghn = jnp.dot(s, wnh_ref[...], preferred_element_type=f32) + bh[:, 2 * D:]
        r = jax.nn.sigmoid(gir + ghr)
        z = jax.nn.sigmoid(giz + ghz)
        n = jnp.tanh(gin + r * ghn)
        outs.append((1.0 - z) * n + z * s)
    return jnp.concatenate(outs, axis=1)


def _upd_body(*refs):
    out_ref = refs[-1]
    out_ref[...] = _gru_compute(*refs[:-1])


_NODE_SPECS = None


def _gru_specs(nb4):
    pp = pl.BlockSpec((2, nb4, 4 * D), lambda i: (0, i, 0))
    sspec = [pl.BlockSpec((nb4, 4 * D), lambda i: (i, 0))]
    wspec = [pl.BlockSpec((D, D), lambda i: (0, 0)) for _ in range(7)]
    bspec = [pl.BlockSpec((1, 3 * D), lambda i: (0, 0)) for _ in range(2)]
    cbspec = [pl.BlockSpec((1, D), lambda i: (0, 0))]
    return [pp, pp] + sspec + wspec[:1] + cbspec + wspec[1:] + bspec


def _update(aggp4, cntp4, s4, root, cb, wsplits, bi, bh):
    nb4 = 1024
    return pl.pallas_call(
        _upd_body,
        grid=(N_NODES // 4 // nb4,),
        in_specs=_gru_specs(nb4),
        out_specs=pl.BlockSpec((nb4, 4 * D), lambda i: (i, 0)),
        out_shape=jax.ShapeDtypeStruct((N_NODES // 4, 4 * D), jnp.float32),
    )(aggp4, cntp4, s4, root, cb, *wsplits, bi, bh)


def _updhead_body(*refs):
    f32 = jnp.float32
    mu_ref, std_ref = refs[-2], refs[-1]
    gru_refs = refs[:13]
    w1_ref, b1_ref, w2_ref, b2_ref, w3_ref, b3_ref, sel_ref = refs[13:20]
    h4 = _gru_compute(*gru_refs)
    sel4 = sel_ref[...]
    mus, stds = [], []
    for q in range(4):
        h = h4[:, q * D:(q + 1) * D]
        t1 = jnp.dot(h, w1_ref[...], preferred_element_type=f32)
        t1 = jnp.maximum(t1 + b1_ref[...], 0.0)
        t2 = jnp.dot(t1, w2_ref[...], preferred_element_type=f32)
        t2 = jnp.maximum(t2 + b2_ref[...], 0.0)
        mix = (jnp.dot(t2, w3_ref[...], preferred_element_type=f32)
               + b3_ref[...])
        sel = sel4[:, q:q + 1]
        lane = lax.broadcasted_iota(jnp.int32, mix.shape, 1)
        oh = (lane == sel).astype(f32)
        mus.append(jnp.sum(mix * oh, axis=1, keepdims=True))
        mm = jnp.mean(mix, axis=1, keepdims=True)
        var = (jnp.sum((mix - mm) * (mix - mm), axis=1, keepdims=True)
               / (MIX_N - 1.0))
        stds.append(jnp.sqrt(var + VAR_EPS))
    mu_ref[...] = jnp.concatenate(mus, axis=1)
    std_ref[...] = jnp.concatenate(stds, axis=1)


def _updheads(aggp4, cntp4, s4, root, cb, wsplits, bi, bh,
              w1, b1, w2, b2, w3, b3, sel4):
    nb4 = 1024
    kd = MIX_N * D
    head_specs = [
        pl.BlockSpec((D, kd), lambda i: (0, 0)),
        pl.BlockSpec((1, kd), lambda i: (0, 0)),
        pl.BlockSpec((kd, kd), lambda i: (0, 0)),
        pl.BlockSpec((1, kd), lambda i: (0, 0)),
        pl.BlockSpec((kd, MIX_N), lambda i: (0, 0)),
        pl.BlockSpec((1, MIX_N), lambda i: (0, 0)),
        pl.BlockSpec((nb4, 4), lambda i: (i, 0)),
    ]
    return pl.pallas_call(
        _updhead_body,
        grid=(N_NODES // 4 // nb4,),
        in_specs=_gru_specs(nb4) + head_specs,
        out_specs=[
            pl.BlockSpec((nb4, 4), lambda i: (i, 0)),
            pl.BlockSpec((nb4, 4), lambda i: (i, 0)),
        ],
        out_shape=[
            jax.ShapeDtypeStruct((N_NODES // 4, 4), jnp.float32),
            jax.ShapeDtypeStruct((N_NODES // 4, 4), jnp.float32),
        ],
    )(aggp4, cntp4, s4, root, cb, *wsplits, bi, bh,
      w1, b1, w2, b2, w3, b3, sel4)


def _gather_body(cur_hbm, src_hbm, out_hbm, idx_v, rows_v, tab_sh, sem):
    c = lax.axis_index("c")
    s = lax.axis_index("s")
    wid = s * 2 + c
    nps = N_NODES // 16
    pltpu.sync_copy(cur_hbm.at[pl.ds(s * nps, nps)], tab_sh.at[pl.ds(s * nps, nps)])
    pltpu.sync_copy(src_hbm.at[pl.ds(wid * NCH, NCH)], idx_v)
    plsc.subcore_barrier()
    copies = []
    for j in range(NCH):
        copies.append(pltpu.async_copy(
            tab_sh.at[idx_v.at[j]], rows_v.at[pl.ds(j * CHUNK, CHUNK)], sem))
    for cp in copies:
        cp.wait()
    pltpu.sync_copy(rows_v, out_hbm.at[pl.ds(wid * EPW, EPW)])


def _sc_gather(cur, src2d):
    mesh = plsc.VectorSubcoreMesh(core_axis_name="c", subcore_axis_name="s")
    return pl.kernel(
        _gather_body,
        out_type=jax.ShapeDtypeStruct((N_EDGES, D), jnp.float32),
        mesh=mesh,
        compiler_params=pltpu.CompilerParams(use_tc_tiling_on_sc=False),
        scratch_types=[
            pltpu.VMEM((NCH, CHUNK), jnp.int32),
            pltpu.VMEM((EPW, D), jnp.float32),
            pltpu.VMEM_SHARED((N_NODES, D), jnp.float32),
            pltpu.SemaphoreType.DMA,
        ],
    )(cur, src2d)


def _scatter_body(msg_hbm, dst_hbm, zero_hbm, out_hbm, idx_v, rows_v, acc_sh, sem):
    c = lax.axis_index("c")
    s = lax.axis_index("s")
    wid = s * 2 + c
    rps = N_NODES // 16
    pltpu.sync_copy(zero_hbm, acc_sh.at[pl.ds(s * rps, rps)])
    pltpu.sync_copy(dst_hbm.at[pl.ds(wid * NCH, NCH)], idx_v)
    pltpu.sync_copy(msg_hbm.at[pl.ds(wid * EPW, EPW)], rows_v)
    plsc.subcore_barrier()
    for j in range(NCH):
        pltpu.sync_copy(rows_v.at[pl.ds(j * CHUNK, CHUNK)],
                        acc_sh.at[idx_v.at[j]], add=True)
    plsc.subcore_barrier()
    pltpu.sync_copy(acc_sh.at[pl.ds(s * rps, rps)],
                    out_hbm.at[c, pl.ds(s * rps, rps)])


def _sc_scatter(msg, dst2d, zero_rows):
    mesh = plsc.VectorSubcoreMesh(core_axis_name="c", subcore_axis_name="s")
    return pl.kernel(
        _scatter_body,
        out_type=jax.ShapeDtypeStruct((2, N_NODES, D), jnp.float32),
        mesh=mesh,
        compiler_params=pltpu.CompilerParams(use_tc_tiling_on_sc=False),
        scratch_types=[
            pltpu.VMEM((NCH, CHUNK), jnp.int32),
            pltpu.VMEM((EPW, D), jnp.float32),
            pltpu.VMEM_SHARED((N_NODES, D), jnp.float32),
            pltpu.SemaphoreType.DMA,
        ],
    )(msg, dst2d, zero_rows)


def kernel(x, edge_index, edge_attr, input_idx, W_embed, b_embed, bn_gamma,
           bn_beta, We1, be1, We2, be2, root, conv_bias, Wih, Whh, bih, bhh,
           mW1, mb1, mW2, mb2, mW3, mb3):
    f32 = jnp.float32
    src2d = edge_index[0].reshape(N_EDGES // CHUNK, CHUNK)
    dst2d = edge_index[1].reshape(N_EDGES // CHUNK, CHUNK)

    wt = W_embed.T
    b2 = b_embed.reshape(1, D)
    g2 = bn_gamma.reshape(1, D)
    bt2 = bn_beta.reshape(1, D)
    w1t = We1.T
    be1r = be1.reshape(1, EL)
    we2p = We2.reshape(D, D, EL).transpose(1, 0, 2).reshape(D * D, EL)
    w2t = we2p.T.astype(jnp.bfloat16)
    be2p = be2.reshape(D, D).T.reshape(1, D * D)
    eye = jnp.eye(D, dtype=jnp.bfloat16)
    tmat = jnp.tile(eye, (1, D))
    gmat = jnp.repeat(eye, D, axis=0)
    cb = conv_bias.reshape(1, D)
    wsplits = (Wih[:D].T, Wih[D:2 * D].T, Wih[2 * D:].T,
               Whh[:D].T, Whh[D:2 * D].T, Whh[2 * D:].T)
    bi = bih.reshape(1, 3 * D)
    bh = bhh.reshape(1, 3 * D)
    kd = MIX_N * D
    w1 = jnp.transpose(mW1, (2, 0, 1)).reshape(D, kd)
    b1 = mb1.reshape(1, kd)
    w2bd = jax.scipy.linalg.block_diag(*[mW2[k].T for k in range(MIX_N)])
    b2h = mb2.reshape(1, kd)
    w3bd = jax.scipy.linalg.block_diag(*[mW3[k].T for k in range(MIX_N)])
    b3h = mb3.reshape(1, MIX_N)
    sel4 = jnp.repeat(jnp.mod(input_idx, MIX_N), MAX_N).reshape(N_NODES // 4, 4)
    sel4 = sel4.astype(jnp.int32)
    zero_rows = jnp.zeros((N_NODES // 16, D), f32)
    ones_rows = jnp.ones((N_EDGES, D), f32)

    s4 = _embed(x.reshape(N_NODES // 4, 4 * D_IN), wt, b2, g2, bt2)
    wg4 = _wgpre(edge_attr.reshape(N_EDGES // 4, 16), w1t, be1r, w2t, be2p)
    cntp4 = _sc_scatter(ones_rows, dst2d,
                        zero_rows).reshape(2, N_NODES // 4, 4 * D)
    for it in range(3):
        cs = _sc_gather(s4.reshape(N_NODES, D), src2d)
        msg4 = _msg(wg4, cs.reshape(N_EDGES // 4, 4 * D), tmat, gmat)
        aggp4 = _sc_scatter(msg4.reshape(N_EDGES, D), dst2d,
                            zero_rows).reshape(2, N_NODES // 4, 4 * D)
        if it < 2:
            s4 = _update(aggp4, cntp4, s4, root, cb, wsplits, bi, bh)
    mu4, std4 = _updheads(aggp4, cntp4, s4, root, cb, wsplits, bi, bh,
                          w1, b1, w2bd, b2h, w3bd, b3h, sel4)
    return (mu4.reshape(N_NODES // MAX_N, MAX_N, 1),
            std4.reshape(N_NODES // MAX_N, MAX_N, 1))
